# baseline (speedup 1.0000x reference)
import functools

import jax
import jax.numpy as jnp
from jax import lax
from jax.experimental import pallas as pl
from jax.experimental.pallas import tpu as pltpu
from jax.experimental.pallas import tpu_sc as plsc

TOKENS = 8192
IN_FEATURES = 1024
HIDDEN = 1024
N_CLASSES = 1000
N_EXPERTS = 8
TOP_K = 2
EXPERT_DIM = 256

TM = 512
EPAD = 128
A_TOT = TOKENS * TOP_K
GTILE = 256
PADTOT = A_TOT + N_EXPERTS * GTILE
NTILES = PADTOT // GTILE
NW = 32
CHUNK_B = A_TOT // NW
ZCHUNK = PADTOT // NW
ROWW = HIDDEN // 2
GCH = 64
TCH = 64
TOK_W = TOKENS // NW

_NT = (((1,), (1,)), ((), ()))

_mesh = plsc.VectorSubcoreMesh(core_axis_name="c", subcore_axis_name="s")


def _wid():
    return lax.axis_index("s") * 2 + lax.axis_index("c")


def _proj_router_body(x_ref, wp_ref, bp_ref, wg_ref,
                      probs_ref, h_ref, ei_ref, wi_ref):
    x = x_ref[...]
    h = lax.dot_general(x, wp_ref[...], _NT, preferred_element_type=jnp.float32)
    h = jnp.maximum(h + bp_ref[...], 0.0)
    h_ref[...] = h.astype(jnp.bfloat16)

    logits = lax.dot_general(h, wg_ref[...], _NT,
                             preferred_element_type=jnp.float32)
    col = lax.broadcasted_iota(jnp.int32, (TM, EPAD), 1)
    logits = jnp.where(col < N_EXPERTS, logits, jnp.float32(-1e30))
    lmax = jnp.max(logits, axis=1, keepdims=True)
    ex = jnp.exp(logits - lmax)
    probs = ex / jnp.sum(ex, axis=1, keepdims=True)
    probs_ref[...] = probs

    w1 = jnp.max(probs, axis=1, keepdims=True)
    i1 = jnp.min(jnp.where(probs == w1, col, EPAD), axis=1, keepdims=True)
    probs2 = jnp.where(col == i1, -1.0, probs)
    w2 = jnp.max(probs2, axis=1, keepdims=True)
    i2 = jnp.min(jnp.where(probs2 == w2, col, EPAD), axis=1, keepdims=True)
    s = w1 + w2
    ei_ref[...] = jnp.where(col == 0, i1, jnp.where(col == 1, i2, 0))
    wi_ref[...] = jnp.where(col == 0, w1 / s, jnp.where(col == 1, w2 / s, 0.0))


@jax.jit
def _stage_a(x, Wp, bp, Wg_pad):
    full = lambda *shape: pl.BlockSpec(shape, lambda i: (0,) * len(shape))
    return pl.pallas_call(
        _proj_router_body,
        grid=(TOKENS // TM,),
        in_specs=[
            pl.BlockSpec((TM, IN_FEATURES), lambda i: (i, 0)),
            full(HIDDEN, IN_FEATURES),
            full(1, HIDDEN),
            full(EPAD, HIDDEN),
        ],
        out_specs=[
            pl.BlockSpec((TM, EPAD), lambda i: (i, 0)),
            pl.BlockSpec((TM, HIDDEN), lambda i: (i, 0)),
            pl.BlockSpec((TM, EPAD), lambda i: (i, 0)),
            pl.BlockSpec((TM, EPAD), lambda i: (i, 0)),
        ],
        out_shape=[
            jax.ShapeDtypeStruct((TOKENS, EPAD), jnp.float32),
            jax.ShapeDtypeStruct((TOKENS, HIDDEN), jnp.bfloat16),
            jax.ShapeDtypeStruct((TOKENS, EPAD), jnp.int32),
            jax.ShapeDtypeStruct((TOKENS, EPAD), jnp.float32),
        ],
    )(x, Wp, bp, Wg_pad)


LPW = CHUNK_B // 16


def _hist(ids_v, lane, zero16):
    def hist_body(i, accs):
        v = plsc.load_gather(ids_v, [lane * LPW + i])
        return tuple(accs[e] + jnp.where(v == e, 1, 0)
                     for e in range(N_EXPERTS))
    return lax.fori_loop(0, LPW, hist_body, (zero16,) * N_EXPERTS)


def _hist_body_k(ei_hbm, cnths_hbm, ids_v, cnt_v, sem):
    wid = _wid()
    base = wid * CHUNK_B
    lane = lax.iota(jnp.int32, 16)
    zero16 = jnp.zeros((16,), jnp.int32)
    pltpu.sync_copy(ei_hbm.at[pl.ds(base, CHUNK_B)], ids_v)
    accs = _hist(ids_v, lane, zero16)
    cnt = zero16
    for e in range(N_EXPERTS):
        acc = accs[e]
        tot_e = acc[0]
        for l in range(1, 16):
            tot_e = tot_e + acc[l]
        cnt = jnp.where(lane == e, jnp.full((16,), tot_e, jnp.int32), cnt)
    cnt_v[...] = cnt
    pltpu.sync_copy(cnt_v, cnths_hbm.at[pl.ds(wid * 16, 16)])


@jax.jit
def _stage_b1(ei):
    return pl.kernel(
        _hist_body_k,
        mesh=_mesh,
        compiler_params=pltpu.CompilerParams(needs_layout_passes=False),
        out_type=jax.ShapeDtypeStruct((NW * 16,), jnp.int32),
        scratch_types=[
            pltpu.VMEM((CHUNK_B,), jnp.int32),
            pltpu.VMEM((16,), jnp.int32),
            pltpu.SemaphoreType.DMA,
        ],
    )(ei)


def _route_body(ei_hbm, wi_hbm, cnths_hbm, gat_hbm, wsort_hbm, pos_hbm,
                texp_hbm, ids_v, w_v, pos_v, tok_v, allcnt_v, texp_v, sem):
    wid = _wid()
    base = wid * CHUNK_B
    lane = lax.iota(jnp.int32, 16)
    zero16 = jnp.zeros((16,), jnp.int32)

    pltpu.sync_copy(ei_hbm.at[pl.ds(base, CHUNK_B)], ids_v)
    pltpu.sync_copy(wi_hbm.at[pl.ds(base, CHUNK_B)], w_v)
    pltpu.sync_copy(cnths_hbm, allcnt_v)
    accs = _hist(ids_v, lane, zero16)

    widv = jnp.full((16,), wid, jnp.int32)
    tot = zero16
    mybase_cnt = zero16
    for w in range(NW):
        row = allcnt_v[pl.ds(w * 16, 16)]
        tot = tot + row
        mybase_cnt = mybase_cnt + jnp.where(
            jnp.full((16,), w, jnp.int32) < widv, row, 0)

    end_scal = []
    lane_base = []
    gs_run = jnp.int32(0)
    for e in range(N_EXPERTS):
        tot_e = tot[e]
        pcnt_e = jnp.bitwise_and(tot_e + (GTILE - 1), ~(GTILE - 1))
        base_e = gs_run + mybase_cnt[e]
        gs_run = gs_run + pcnt_e
        end_scal.append(gs_run)
        vec = zero16
        run_s = base_e
        acc = accs[e]
        for l in range(16):
            vec = jnp.where(lane == l, jnp.full((16,), run_s, jnp.int32), vec)
            run_s = run_s + acc[l]
        lane_base.append(vec)


    def pos_body(i, rs):
        idxv = lane * LPW + i
        v = plsc.load_gather(ids_v, [idxv])
        posv = zero16
        new = []
        for e in range(N_EXPERTS):
            m = v == e
            posv = jnp.where(m, lane_base[e] + rs[e], posv)
            new.append(rs[e] + jnp.where(m, 1, 0))
        plsc.store_scatter(pos_v, [idxv], posv)
        return tuple(new)
    lax.fori_loop(0, LPW, pos_body, (zero16,) * N_EXPERTS)
    for j in range(CHUNK_B // 16):
        tok_v[pl.ds(j * 16, 16)] = jnp.right_shift(
            jnp.full((16,), base + j * 16, jnp.int32) + lane, 1)

    pltpu.async_copy(tok_v, gat_hbm.at[pos_v], sem).wait()
    pltpu.async_copy(w_v, wsort_hbm.at[pos_v], sem).wait()
    pltpu.sync_copy(pos_v, pos_hbm.at[pl.ds(base, CHUNK_B)])

    @pl.when(wid == 0)
    def _():
        for j in range(8):
            jv = (jnp.full((16,), j * 16, jnp.int32) + lane) * GTILE
            t = zero16
            for e in range(N_EXPERTS):
                t = t + jnp.where(
                    jv >= jnp.full((16,), end_scal[e], jnp.int32), 1, 0)
            texp_v[pl.ds(j * 16, 16)] = jnp.minimum(t, N_EXPERTS - 1)
        pltpu.sync_copy(texp_v, texp_hbm)


@jax.jit
def _stage_b2(ei, wi, cnths):
    return pl.kernel(
        _route_body,
        mesh=_mesh,
        compiler_params=pltpu.CompilerParams(needs_layout_passes=False),
        out_type=[
            jax.ShapeDtypeStruct((PADTOT,), jnp.int32),
            jax.ShapeDtypeStruct((PADTOT,), jnp.float32),
            jax.ShapeDtypeStruct((A_TOT,), jnp.int32),
            jax.ShapeDtypeStruct((128,), jnp.int32),
        ],
        scratch_types=[
            pltpu.VMEM((CHUNK_B,), jnp.int32),
            pltpu.VMEM((CHUNK_B,), jnp.float32),
            pltpu.VMEM((CHUNK_B,), jnp.int32),
            pltpu.VMEM((CHUNK_B,), jnp.int32),
            pltpu.VMEM((NW * 16,), jnp.int32),
            pltpu.VMEM((128,), jnp.int32),
            pltpu.SemaphoreType.DMA,
        ],
    )(ei, wi, cnths)


def _gather_body(h32_hbm, gat_hbm, hs_hbm, idx_v, rows_v, sem):
    base = _wid() * ZCHUNK

    def body(c, _):
        off = base + c * GCH
        pltpu.sync_copy(gat_hbm.at[pl.ds(off, GCH)], idx_v)
        for j in range(GCH // 16):
            v = idx_v[pl.ds(j * 16, 16)]
            idx_v[pl.ds(j * 16, 16)] = jnp.clip(v, 0, TOKENS - 1)
        pltpu.async_copy(h32_hbm.at[idx_v], rows_v, sem).wait()
        pltpu.sync_copy(rows_v, hs_hbm.at[pl.ds(off, GCH)])
        return 0
    lax.fori_loop(0, ZCHUNK // GCH, body, 0)


@jax.jit
def _stage_c(h32, gat):
    return pl.kernel(
        _gather_body,
        mesh=_mesh,
        compiler_params=pltpu.CompilerParams(needs_layout_passes=False),
        out_type=jax.ShapeDtypeStruct((PADTOT, ROWW), jnp.int32),
        scratch_types=[
            pltpu.VMEM((GCH,), jnp.int32),
            pltpu.VMEM((GCH, ROWW), jnp.int32),
            pltpu.SemaphoreType.DMA,
        ],
    )(h32, gat)


def _ffn_body(texp_ref, hs_ref, w1_ref, b1_ref, w2_ref, b2_ref, ws_ref,
              ys_ref):
    hsb = hs_ref[...]
    hid = lax.dot_general(hsb, w1_ref[0], _NT,
                          preferred_element_type=jnp.float32)
    hid = jnp.maximum(hid + b1_ref[0], 0.0)
    out = lax.dot_general(hid.astype(jnp.bfloat16), w2_ref[0], _NT,
                          preferred_element_type=jnp.float32)
    out = out + b2_ref[0]
    ri = lax.broadcasted_iota(jnp.int32, (GTILE, GTILE), 0)
    ci = lax.broadcasted_iota(jnp.int32, (GTILE, GTILE), 1)
    diag_w = jnp.where(ri == ci, jnp.broadcast_to(ws_ref[0], (GTILE, GTILE)),
                       0.0)
    ys = lax.dot_general(diag_w, out, (((1,), (0,)), ((), ())),
                         preferred_element_type=jnp.float32)
    ys_ref[...] = ys.astype(jnp.bfloat16)


@jax.jit
def _stage_d(texp, hs_bf, W1b, b1, W2b, b2, ws2d):
    grid_spec = pltpu.PrefetchScalarGridSpec(
        num_scalar_prefetch=1,
        grid=(NTILES,),
        in_specs=[
            pl.BlockSpec((GTILE, HIDDEN), lambda i, t: (i, 0)),
            pl.BlockSpec((1, EXPERT_DIM, HIDDEN), lambda i, t: (t[i], 0, 0)),
            pl.BlockSpec((1, 1, EXPERT_DIM), lambda i, t: (t[i], 0, 0)),
            pl.BlockSpec((1, HIDDEN, EXPERT_DIM), lambda i, t: (t[i], 0, 0)),
            pl.BlockSpec((1, 1, HIDDEN), lambda i, t: (t[i], 0, 0)),
            pl.BlockSpec((1, 1, GTILE), lambda i, t: (i, 0, 0)),
        ],
        out_specs=pl.BlockSpec((GTILE, HIDDEN), lambda i, t: (i, 0)),
    )
    return pl.pallas_call(
        _ffn_body,
        grid_spec=grid_spec,
        out_shape=jax.ShapeDtypeStruct((PADTOT, HIDDEN), jnp.bfloat16),
    )(texp, hs_bf, W1b, b1, W2b, b2, ws2d)


def _combine_body(ys_hbm, pos_hbm, h2_hbm, posc_v, idx0_v, idx1_v,
                  y0_v, y1_v, out_v, sem):
    wid = _wid()
    lane = lax.iota(jnp.int32, 16)

    def chunk_body(c, _):
        tok0 = wid * TOK_W + c * TCH
        pltpu.sync_copy(pos_hbm.at[pl.ds(tok0 * 2, TCH * 2)], posc_v)
        for j in range(TCH // 16):
            tl = jnp.full((16,), j * 16, jnp.int32) + lane
            idx0_v[pl.ds(j * 16, 16)] = plsc.load_gather(posc_v, [tl * 2])
            idx1_v[pl.ds(j * 16, 16)] = plsc.load_gather(posc_v, [tl * 2 + 1])
        pltpu.async_copy(ys_hbm.at[idx0_v], y0_v, sem).wait()
        pltpu.async_copy(ys_hbm.at[idx1_v], y1_v, sem).wait()

        def add_body(r, _):
            for cc in range(ROWW // 16):
                a = plsc.bitcast(y0_v[r, pl.ds(cc * 16, 16)], jnp.bfloat16)
                b = plsc.bitcast(y1_v[r, pl.ds(cc * 16, 16)], jnp.bfloat16)
                rr = jnp.maximum(a + b, jnp.bfloat16(0))
                out_v[r, pl.ds(cc * 16, 16)] = plsc.bitcast(rr, jnp.int32)
            return 0
        lax.fori_loop(0, TCH, add_body, 0)
        pltpu.sync_copy(out_v, h2_hbm.at[pl.ds(tok0, TCH)])
        return 0
    lax.fori_loop(0, TOK_W // TCH, chunk_body, 0)


@jax.jit
def _stage_e(ys32, pos):
    return pl.kernel(
        _combine_body,
        mesh=_mesh,
        compiler_params=pltpu.CompilerParams(needs_layout_passes=False),
        out_type=jax.ShapeDtypeStruct((TOKENS, ROWW), jnp.int32),
        scratch_types=[
            pltpu.VMEM((TCH * 2,), jnp.int32),
            pltpu.VMEM((TCH,), jnp.int32),
            pltpu.VMEM((TCH,), jnp.int32),
            pltpu.VMEM((TCH, ROWW), jnp.int32),
            pltpu.VMEM((TCH, ROWW), jnp.int32),
            pltpu.VMEM((TCH, ROWW), jnp.int32),
            pltpu.SemaphoreType.DMA,
        ],
    )(ys32, pos)


def _cls_body(h2_ref, wc_ref, bc_ref, cls_ref):
    cls = lax.dot_general(h2_ref[...], wc_ref[...], _NT,
                          preferred_element_type=jnp.float32)
    cls_ref[...] = cls + bc_ref[...]


@jax.jit
def _stage_f(h2_bf, Wcb, bc):
    full = lambda *shape: pl.BlockSpec(shape, lambda i: (0,) * len(shape))
    return pl.pallas_call(
        _cls_body,
        grid=(TOKENS // TM,),
        in_specs=[
            pl.BlockSpec((TM, HIDDEN), lambda i: (i, 0)),
            full(N_CLASSES, HIDDEN),
            full(1, N_CLASSES),
        ],
        out_specs=pl.BlockSpec((TM, N_CLASSES), lambda i: (i, 0)),
        out_shape=jax.ShapeDtypeStruct((TOKENS, N_CLASSES), jnp.float32),
    )(h2_bf, Wcb, bc)


def kernel(x, Wp, bp, Wg, W1, b1, W2, b2, Wc, bc):
    Wg_pad = jnp.zeros((EPAD, HIDDEN), jnp.float32).at[:N_EXPERTS].set(Wg)
    probs_pad, h_bf, ei_pad, wi_pad = _stage_a(x, Wp, bp[None, :], Wg_pad)
    ei = ei_pad[:, :TOP_K].reshape(-1)
    wi = wi_pad[:, :TOP_K].reshape(-1)
    cnths = _stage_b1(ei)
    gat, wsort, pos, texp = _stage_b2(ei, wi, cnths)
    h32 = lax.bitcast_convert_type(
        h_bf.reshape(TOKENS, ROWW, 2), jnp.int32)
    hs32 = _stage_c(h32, gat)
    hs_bf = lax.bitcast_convert_type(hs32, jnp.bfloat16).reshape(
        PADTOT, HIDDEN)
    ys = _stage_d(texp, hs_bf, W1.astype(jnp.bfloat16),
                  b1.reshape(N_EXPERTS, 1, EXPERT_DIM),
                  W2.astype(jnp.bfloat16),
                  b2.reshape(N_EXPERTS, 1, HIDDEN),
                  wsort.reshape(NTILES, 1, GTILE))
    ys32 = lax.bitcast_convert_type(
        ys.reshape(PADTOT, ROWW, 2), jnp.int32)
    h2_32 = _stage_e(ys32, pos)
    h2_bf = lax.bitcast_convert_type(h2_32, jnp.bfloat16).reshape(
        TOKENS, HIDDEN)
    cls = _stage_f(h2_bf, Wc.astype(jnp.bfloat16), bc[None, :])
    return cls, probs_pad[:, :N_EXPERTS]


# speedup vs baseline: 2.7495x; 2.7495x over previous
import functools

import jax
import jax.numpy as jnp
from jax import lax
from jax.experimental import pallas as pl
from jax.experimental.pallas import tpu as pltpu
from jax.experimental.pallas import tpu_sc as plsc

TOKENS = 8192
IN_FEATURES = 1024
HIDDEN = 1024
N_CLASSES = 1000
N_EXPERTS = 8
TOP_K = 2
EXPERT_DIM = 256

TM = 512
EPAD = 128
A_TOT = TOKENS * TOP_K
GTILE = 256
PADTOT = A_TOT + N_EXPERTS * GTILE
NTILES = PADTOT // GTILE
NW = 32
CHUNK_B = A_TOT // NW
ZCHUNK = PADTOT // NW
ROWW = HIDDEN
GCH = 64
TCH = 32
TOK_W = TOKENS // NW

_NT = (((1,), (1,)), ((), ()))

_mesh = plsc.VectorSubcoreMesh(core_axis_name="c", subcore_axis_name="s")


def _wid():
    return lax.axis_index("s") * 2 + lax.axis_index("c")


def _proj_router_body(x_ref, wp_ref, bp_ref, wg_ref,
                      probs_ref, h_ref, ei_ref, wi_ref):
    x = x_ref[...]
    h = lax.dot_general(x, wp_ref[...], _NT, preferred_element_type=jnp.float32)
    h = jnp.maximum(h + bp_ref[...], 0.0)
    h_ref[...] = h

    logits = lax.dot_general(h, wg_ref[...], _NT,
                             preferred_element_type=jnp.float32)
    col = lax.broadcasted_iota(jnp.int32, (TM, EPAD), 1)
    logits = jnp.where(col < N_EXPERTS, logits, jnp.float32(-1e30))
    lmax = jnp.max(logits, axis=1, keepdims=True)
    ex = jnp.exp(logits - lmax)
    probs = ex / jnp.sum(ex, axis=1, keepdims=True)
    probs_ref[...] = probs

    w1 = jnp.max(probs, axis=1, keepdims=True)
    i1 = jnp.min(jnp.where(probs == w1, col, EPAD), axis=1, keepdims=True)
    probs2 = jnp.where(col == i1, -1.0, probs)
    w2 = jnp.max(probs2, axis=1, keepdims=True)
    i2 = jnp.min(jnp.where(probs2 == w2, col, EPAD), axis=1, keepdims=True)
    s = w1 + w2
    ei_ref[...] = jnp.where(col == 0, i1, jnp.where(col == 1, i2, 0))
    wi_ref[...] = jnp.where(col == 0, w1 / s, jnp.where(col == 1, w2 / s, 0.0))


@jax.jit
def _stage_a(x, Wp, bp, Wg_pad):
    full = lambda *shape: pl.BlockSpec(shape, lambda i: (0,) * len(shape))
    return pl.pallas_call(
        _proj_router_body,
        grid=(TOKENS // TM,),
        in_specs=[
            pl.BlockSpec((TM, IN_FEATURES), lambda i: (i, 0)),
            full(HIDDEN, IN_FEATURES),
            full(1, HIDDEN),
            full(EPAD, HIDDEN),
        ],
        out_specs=[
            pl.BlockSpec((TM, EPAD), lambda i: (i, 0)),
            pl.BlockSpec((TM, HIDDEN), lambda i: (i, 0)),
            pl.BlockSpec((TM, EPAD), lambda i: (i, 0)),
            pl.BlockSpec((TM, EPAD), lambda i: (i, 0)),
        ],
        out_shape=[
            jax.ShapeDtypeStruct((TOKENS, EPAD), jnp.float32),
            jax.ShapeDtypeStruct((TOKENS, HIDDEN), jnp.float32),
            jax.ShapeDtypeStruct((TOKENS, EPAD), jnp.int32),
            jax.ShapeDtypeStruct((TOKENS, EPAD), jnp.float32),
        ],
    )(x, Wp, bp, Wg_pad)


LPW = CHUNK_B // 16


def _hist(ids_v, lane, zero16):
    def hist_body(i, accs):
        v = plsc.load_gather(ids_v, [lane * LPW + i])
        return tuple(accs[e] + jnp.where(v == e, 1, 0)
                     for e in range(N_EXPERTS))
    return lax.fori_loop(0, LPW, hist_body, (zero16,) * N_EXPERTS)


def _hist_body_k(ei_hbm, cnths_hbm, ids_v, cnt_v, sem):
    wid = _wid()
    base = wid * CHUNK_B
    lane = lax.iota(jnp.int32, 16)
    zero16 = jnp.zeros((16,), jnp.int32)
    pltpu.sync_copy(ei_hbm.at[pl.ds(base, CHUNK_B)], ids_v)
    accs = _hist(ids_v, lane, zero16)
    cnt = zero16
    for e in range(N_EXPERTS):
        acc = accs[e]
        tot_e = acc[0]
        for l in range(1, 16):
            tot_e = tot_e + acc[l]
        cnt = jnp.where(lane == e, jnp.full((16,), tot_e, jnp.int32), cnt)
    cnt_v[...] = cnt
    pltpu.sync_copy(cnt_v, cnths_hbm.at[pl.ds(wid * 16, 16)])


@jax.jit
def _stage_b1(ei):
    return pl.kernel(
        _hist_body_k,
        mesh=_mesh,
        compiler_params=pltpu.CompilerParams(needs_layout_passes=False),
        out_type=jax.ShapeDtypeStruct((NW * 16,), jnp.int32),
        scratch_types=[
            pltpu.VMEM((CHUNK_B,), jnp.int32),
            pltpu.VMEM((16,), jnp.int32),
            pltpu.SemaphoreType.DMA,
        ],
    )(ei)


def _route_body(ei_hbm, wi_hbm, cnths_hbm, gat_hbm, wsort_hbm, pos_hbm,
                texp_hbm, ids_v, w_v, pos_v, tok_v, allcnt_v, texp_v, sem):
    wid = _wid()
    base = wid * CHUNK_B
    lane = lax.iota(jnp.int32, 16)
    zero16 = jnp.zeros((16,), jnp.int32)

    pltpu.sync_copy(ei_hbm.at[pl.ds(base, CHUNK_B)], ids_v)
    pltpu.sync_copy(wi_hbm.at[pl.ds(base, CHUNK_B)], w_v)
    pltpu.sync_copy(cnths_hbm, allcnt_v)
    accs = _hist(ids_v, lane, zero16)

    widv = jnp.full((16,), wid, jnp.int32)
    tot = zero16
    mybase_cnt = zero16
    for w in range(NW):
        row = allcnt_v[pl.ds(w * 16, 16)]
        tot = tot + row
        mybase_cnt = mybase_cnt + jnp.where(
            jnp.full((16,), w, jnp.int32) < widv, row, 0)

    end_scal = []
    lane_base = []
    gs_run = jnp.int32(0)
    for e in range(N_EXPERTS):
        tot_e = tot[e]
        pcnt_e = jnp.bitwise_and(tot_e + (GTILE - 1), ~(GTILE - 1))
        base_e = gs_run + mybase_cnt[e]
        gs_run = gs_run + pcnt_e
        end_scal.append(gs_run)
        vec = zero16
        run_s = base_e
        acc = accs[e]
        for l in range(16):
            vec = jnp.where(lane == l, jnp.full((16,), run_s, jnp.int32), vec)
            run_s = run_s + acc[l]
        lane_base.append(vec)


    def pos_body(i, rs):
        idxv = lane * LPW + i
        v = plsc.load_gather(ids_v, [idxv])
        posv = zero16
        new = []
        for e in range(N_EXPERTS):
            m = v == e
            posv = jnp.where(m, lane_base[e] + rs[e], posv)
            new.append(rs[e] + jnp.where(m, 1, 0))
        plsc.store_scatter(pos_v, [idxv], posv)
        return tuple(new)
    lax.fori_loop(0, LPW, pos_body, (zero16,) * N_EXPERTS)
    for j in range(CHUNK_B // 16):
        tok_v[pl.ds(j * 16, 16)] = jnp.right_shift(
            jnp.full((16,), base + j * 16, jnp.int32) + lane, 1)

    pltpu.async_copy(tok_v, gat_hbm.at[pos_v], sem).wait()
    pltpu.async_copy(w_v, wsort_hbm.at[pos_v], sem).wait()
    pltpu.sync_copy(pos_v, pos_hbm.at[pl.ds(base, CHUNK_B)])

    @pl.when(wid == 0)
    def _():
        for j in range(8):
            jv = (jnp.full((16,), j * 16, jnp.int32) + lane) * GTILE
            t = zero16
            for e in range(N_EXPERTS):
                t = t + jnp.where(
                    jv >= jnp.full((16,), end_scal[e], jnp.int32), 1, 0)
            texp_v[pl.ds(j * 16, 16)] = jnp.minimum(t, N_EXPERTS - 1)
        pltpu.sync_copy(texp_v, texp_hbm)


@jax.jit
def _stage_b2(ei, wi, cnths):
    return pl.kernel(
        _route_body,
        mesh=_mesh,
        compiler_params=pltpu.CompilerParams(needs_layout_passes=False),
        out_type=[
            jax.ShapeDtypeStruct((PADTOT,), jnp.int32),
            jax.ShapeDtypeStruct((PADTOT,), jnp.float32),
            jax.ShapeDtypeStruct((A_TOT,), jnp.int32),
            jax.ShapeDtypeStruct((128,), jnp.int32),
        ],
        scratch_types=[
            pltpu.VMEM((CHUNK_B,), jnp.int32),
            pltpu.VMEM((CHUNK_B,), jnp.float32),
            pltpu.VMEM((CHUNK_B,), jnp.int32),
            pltpu.VMEM((CHUNK_B,), jnp.int32),
            pltpu.VMEM((NW * 16,), jnp.int32),
            pltpu.VMEM((128,), jnp.int32),
            pltpu.SemaphoreType.DMA,
        ],
    )(ei, wi, cnths)


def _gather_body(h32_hbm, gat_hbm, hs_hbm, idx_v, rows_v, sem):
    base = _wid() * ZCHUNK

    def body(c, _):
        off = base + c * GCH
        pltpu.sync_copy(gat_hbm.at[pl.ds(off, GCH)], idx_v)
        for j in range(GCH // 16):
            v = idx_v[pl.ds(j * 16, 16)]
            idx_v[pl.ds(j * 16, 16)] = jnp.clip(v, 0, TOKENS - 1)
        pltpu.async_copy(h32_hbm.at[idx_v], rows_v, sem).wait()
        pltpu.sync_copy(rows_v, hs_hbm.at[pl.ds(off, GCH)])
        return 0
    lax.fori_loop(0, ZCHUNK // GCH, body, 0)


@jax.jit
def _stage_c(h32, gat):
    return pl.kernel(
        _gather_body,
        mesh=_mesh,
        compiler_params=pltpu.CompilerParams(needs_layout_passes=False),
        out_type=jax.ShapeDtypeStruct((PADTOT, ROWW), jnp.float32),
        scratch_types=[
            pltpu.VMEM((GCH,), jnp.int32),
            pltpu.VMEM((GCH, ROWW), jnp.float32),
            pltpu.SemaphoreType.DMA,
        ],
    )(h32, gat)


def _ffn_body(texp_ref, hs_ref, w1_ref, b1_ref, w2_ref, b2_ref, ws_ref,
              ys_ref):
    hsb = hs_ref[...].astype(jnp.bfloat16)
    hid = lax.dot_general(hsb, w1_ref[0], _NT,
                          preferred_element_type=jnp.float32)
    hid = jnp.maximum(hid + b1_ref[0], 0.0)
    out = lax.dot_general(hid.astype(jnp.bfloat16), w2_ref[0], _NT,
                          preferred_element_type=jnp.float32)
    out = out + b2_ref[0]
    ri = lax.broadcasted_iota(jnp.int32, (GTILE, GTILE), 0)
    ci = lax.broadcasted_iota(jnp.int32, (GTILE, GTILE), 1)
    diag_w = jnp.where(ri == ci, jnp.broadcast_to(ws_ref[0], (GTILE, GTILE)),
                       0.0)
    ys = lax.dot_general(diag_w, out, (((1,), (0,)), ((), ())),
                         preferred_element_type=jnp.float32)
    ys_ref[...] = ys


@jax.jit
def _stage_d(texp, hs_bf, W1b, b1, W2b, b2, ws2d):
    grid_spec = pltpu.PrefetchScalarGridSpec(
        num_scalar_prefetch=1,
        grid=(NTILES,),
        in_specs=[
            pl.BlockSpec((GTILE, HIDDEN), lambda i, t: (i, 0)),
            pl.BlockSpec((1, EXPERT_DIM, HIDDEN), lambda i, t: (t[i], 0, 0)),
            pl.BlockSpec((1, 1, EXPERT_DIM), lambda i, t: (t[i], 0, 0)),
            pl.BlockSpec((1, HIDDEN, EXPERT_DIM), lambda i, t: (t[i], 0, 0)),
            pl.BlockSpec((1, 1, HIDDEN), lambda i, t: (t[i], 0, 0)),
            pl.BlockSpec((1, 1, GTILE), lambda i, t: (i, 0, 0)),
        ],
        out_specs=pl.BlockSpec((GTILE, HIDDEN), lambda i, t: (i, 0)),
    )
    return pl.pallas_call(
        _ffn_body,
        grid_spec=grid_spec,
        out_shape=jax.ShapeDtypeStruct((PADTOT, HIDDEN), jnp.float32),
    )(texp, hs_bf, W1b, b1, W2b, b2, ws2d)


def _combine_body(ys_hbm, pos_hbm, h2_hbm, posc_v, idx0_v, idx1_v,
                  y0_v, y1_v, out_v, sem):
    wid = _wid()
    lane = lax.iota(jnp.int32, 16)

    def chunk_body(c, _):
        tok0 = wid * TOK_W + c * TCH
        pltpu.sync_copy(pos_hbm.at[pl.ds(tok0 * 2, TCH * 2)], posc_v)
        for j in range(TCH // 16):
            tl = jnp.full((16,), j * 16, jnp.int32) + lane
            idx0_v[pl.ds(j * 16, 16)] = plsc.load_gather(posc_v, [tl * 2])
            idx1_v[pl.ds(j * 16, 16)] = plsc.load_gather(posc_v, [tl * 2 + 1])
        pltpu.async_copy(ys_hbm.at[idx0_v], y0_v, sem).wait()
        pltpu.async_copy(ys_hbm.at[idx1_v], y1_v, sem).wait()

        def add_body(r, _):
            for cc in range(ROWW // 16):
                a = y0_v[r, pl.ds(cc * 16, 16)]
                b = y1_v[r, pl.ds(cc * 16, 16)]
                out_v[r, pl.ds(cc * 16, 16)] = jnp.maximum(a + b, 0.0)
            return 0
        lax.fori_loop(0, TCH, add_body, 0)
        pltpu.sync_copy(out_v, h2_hbm.at[pl.ds(tok0, TCH)])
        return 0
    lax.fori_loop(0, TOK_W // TCH, chunk_body, 0)


@jax.jit
def _stage_e(ys32, pos):
    return pl.kernel(
        _combine_body,
        mesh=_mesh,
        compiler_params=pltpu.CompilerParams(needs_layout_passes=False),
        out_type=jax.ShapeDtypeStruct((TOKENS, ROWW), jnp.float32),
        scratch_types=[
            pltpu.VMEM((TCH * 2,), jnp.int32),
            pltpu.VMEM((TCH,), jnp.int32),
            pltpu.VMEM((TCH,), jnp.int32),
            pltpu.VMEM((TCH, ROWW), jnp.float32),
            pltpu.VMEM((TCH, ROWW), jnp.float32),
            pltpu.VMEM((TCH, ROWW), jnp.float32),
            pltpu.SemaphoreType.DMA,
        ],
    )(ys32, pos)


def _cls_body(h2_ref, wc_ref, bc_ref, cls_ref):
    cls = lax.dot_general(h2_ref[...].astype(jnp.bfloat16), wc_ref[...], _NT,
                          preferred_element_type=jnp.float32)
    cls_ref[...] = cls + bc_ref[...]


@jax.jit
def _stage_f(h2_bf, Wcb, bc):
    full = lambda *shape: pl.BlockSpec(shape, lambda i: (0,) * len(shape))
    return pl.pallas_call(
        _cls_body,
        grid=(TOKENS // TM,),
        in_specs=[
            pl.BlockSpec((TM, HIDDEN), lambda i: (i, 0)),
            full(N_CLASSES, HIDDEN),
            full(1, N_CLASSES),
        ],
        out_specs=pl.BlockSpec((TM, N_CLASSES), lambda i: (i, 0)),
        out_shape=jax.ShapeDtypeStruct((TOKENS, N_CLASSES), jnp.float32),
    )(h2_bf, Wcb, bc)


def kernel(x, Wp, bp, Wg, W1, b1, W2, b2, Wc, bc):
    Wg_pad = jnp.zeros((EPAD, HIDDEN), jnp.float32).at[:N_EXPERTS].set(Wg)
    probs_pad, h_f, ei_pad, wi_pad = _stage_a(x, Wp, bp[None, :], Wg_pad)
    ei = ei_pad[:, :TOP_K].reshape(-1)
    wi = wi_pad[:, :TOP_K].reshape(-1)
    cnths = _stage_b1(ei)
    gat, wsort, pos, texp = _stage_b2(ei, wi, cnths)
    hs = _stage_c(h_f, gat)
    ys = _stage_d(texp, hs, W1.astype(jnp.bfloat16),
                  b1.reshape(N_EXPERTS, 1, EXPERT_DIM),
                  W2.astype(jnp.bfloat16),
                  b2.reshape(N_EXPERTS, 1, HIDDEN),
                  wsort.reshape(NTILES, 1, GTILE))
    h2 = _stage_e(ys, pos)
    cls = _stage_f(h2, Wc.astype(jnp.bfloat16), bc[None, :])
    return cls, probs_pad[:, :N_EXPERTS]


# speedup vs baseline: 3.6377x; 1.3230x over previous
import functools

import jax
import jax.numpy as jnp
from jax import lax
from jax.experimental import pallas as pl
from jax.experimental.pallas import tpu as pltpu
from jax.experimental.pallas import tpu_sc as plsc

TOKENS = 8192
IN_FEATURES = 1024
HIDDEN = 1024
N_CLASSES = 1000
N_EXPERTS = 8
TOP_K = 2
EXPERT_DIM = 256

TM = 512
EPAD = 128
A_TOT = TOKENS * TOP_K
GTILE = 256
PADTOT = A_TOT + N_EXPERTS * GTILE
NTILES = PADTOT // GTILE
NW = 32
CHUNK_B = A_TOT // NW
ZCHUNK = PADTOT // NW
ROWW = HIDDEN
GCH = 64
TCH = 32
TOK_W = TOKENS // NW

_NT = (((1,), (1,)), ((), ()))

_mesh = plsc.VectorSubcoreMesh(core_axis_name="c", subcore_axis_name="s")


def _wid():
    return lax.axis_index("s") * 2 + lax.axis_index("c")


def _proj_router_body(x_ref, wp_ref, bp_ref, wg_ref,
                      probs_ref, h_ref, ei_ref, wi_ref):
    x = x_ref[...]
    h = lax.dot_general(x, wp_ref[...], _NT, preferred_element_type=jnp.float32)
    h = jnp.maximum(h + bp_ref[...], 0.0)
    h_ref[...] = h

    logits = lax.dot_general(h, wg_ref[...], _NT,
                             preferred_element_type=jnp.float32)
    col = lax.broadcasted_iota(jnp.int32, (TM, EPAD), 1)
    logits = jnp.where(col < N_EXPERTS, logits, jnp.float32(-1e30))
    lmax = jnp.max(logits, axis=1, keepdims=True)
    ex = jnp.exp(logits - lmax)
    probs = ex / jnp.sum(ex, axis=1, keepdims=True)
    probs_ref[...] = probs

    w1 = jnp.max(probs, axis=1, keepdims=True)
    i1 = jnp.min(jnp.where(probs == w1, col, EPAD), axis=1, keepdims=True)
    probs2 = jnp.where(col == i1, -1.0, probs)
    w2 = jnp.max(probs2, axis=1, keepdims=True)
    i2 = jnp.min(jnp.where(probs2 == w2, col, EPAD), axis=1, keepdims=True)
    s = w1 + w2
    ei_ref[...] = jnp.where(col == 0, i1, jnp.where(col == 1, i2, 0))
    wi_ref[...] = jnp.where(col == 0, w1 / s, jnp.where(col == 1, w2 / s, 0.0))


@jax.jit
def _stage_a(x, Wp, bp, Wg_pad):
    full = lambda *shape: pl.BlockSpec(shape, lambda i: (0,) * len(shape))
    return pl.pallas_call(
        _proj_router_body,
        grid=(TOKENS // TM,),
        in_specs=[
            pl.BlockSpec((TM, IN_FEATURES), lambda i: (i, 0)),
            full(HIDDEN, IN_FEATURES),
            full(1, HIDDEN),
            full(EPAD, HIDDEN),
        ],
        out_specs=[
            pl.BlockSpec((TM, EPAD), lambda i: (i, 0)),
            pl.BlockSpec((TM, HIDDEN), lambda i: (i, 0)),
            pl.BlockSpec((TM, EPAD), lambda i: (i, 0)),
            pl.BlockSpec((TM, EPAD), lambda i: (i, 0)),
        ],
        out_shape=[
            jax.ShapeDtypeStruct((TOKENS, EPAD), jnp.float32),
            jax.ShapeDtypeStruct((TOKENS, HIDDEN), jnp.float32),
            jax.ShapeDtypeStruct((TOKENS, EPAD), jnp.int32),
            jax.ShapeDtypeStruct((TOKENS, EPAD), jnp.float32),
        ],
    )(x, Wp, bp, Wg_pad)


LPW = CHUNK_B // 16


def _hist(ids_v, lane, zero16):
    def hist_body(i, accs):
        v = plsc.load_gather(ids_v, [lane * LPW + i])
        return tuple(accs[e] + jnp.where(v == e, 1, 0)
                     for e in range(N_EXPERTS))
    return lax.fori_loop(0, LPW, hist_body, (zero16,) * N_EXPERTS)


def _hist_body_k(ei_hbm, cnths_hbm, ids_v, cnt_v, sem):
    wid = _wid()
    base = wid * CHUNK_B
    lane = lax.iota(jnp.int32, 16)
    zero16 = jnp.zeros((16,), jnp.int32)
    pltpu.sync_copy(ei_hbm.at[pl.ds(base, CHUNK_B)], ids_v)
    accs = _hist(ids_v, lane, zero16)
    cnt = zero16
    for e in range(N_EXPERTS):
        acc = accs[e]
        tot_e = acc[0]
        for l in range(1, 16):
            tot_e = tot_e + acc[l]
        cnt = jnp.where(lane == e, jnp.full((16,), tot_e, jnp.int32), cnt)
    cnt_v[...] = cnt
    pltpu.sync_copy(cnt_v, cnths_hbm.at[pl.ds(wid * 16, 16)])


@jax.jit
def _stage_b1(ei):
    return pl.kernel(
        _hist_body_k,
        mesh=_mesh,
        compiler_params=pltpu.CompilerParams(needs_layout_passes=False),
        out_type=jax.ShapeDtypeStruct((NW * 16,), jnp.int32),
        scratch_types=[
            pltpu.VMEM((CHUNK_B,), jnp.int32),
            pltpu.VMEM((16,), jnp.int32),
            pltpu.SemaphoreType.DMA,
        ],
    )(ei)


def _route_body(ei_hbm, cnths_hbm, pos_hbm, texp_hbm,
                ids_v, pos_v, allcnt_v, texp_v, sem):
    wid = _wid()
    base = wid * CHUNK_B
    lane = lax.iota(jnp.int32, 16)
    zero16 = jnp.zeros((16,), jnp.int32)

    pltpu.sync_copy(ei_hbm.at[pl.ds(base, CHUNK_B)], ids_v)
    pltpu.sync_copy(cnths_hbm, allcnt_v)
    accs = _hist(ids_v, lane, zero16)

    widv = jnp.full((16,), wid, jnp.int32)
    tot = zero16
    mybase_cnt = zero16
    for w in range(NW):
        row = allcnt_v[pl.ds(w * 16, 16)]
        tot = tot + row
        mybase_cnt = mybase_cnt + jnp.where(
            jnp.full((16,), w, jnp.int32) < widv, row, 0)

    end_scal = []
    lane_base = []
    gs_run = jnp.int32(0)
    for e in range(N_EXPERTS):
        tot_e = tot[e]
        pcnt_e = jnp.bitwise_and(tot_e + (GTILE - 1), ~(GTILE - 1))
        base_e = gs_run + mybase_cnt[e]
        gs_run = gs_run + pcnt_e
        end_scal.append(gs_run)
        vec = zero16
        run_s = base_e
        acc = accs[e]
        for l in range(16):
            vec = jnp.where(lane == l, jnp.full((16,), run_s, jnp.int32), vec)
            run_s = run_s + acc[l]
        lane_base.append(vec)

    def pos_body(i, rs):
        idxv = lane * LPW + i
        v = plsc.load_gather(ids_v, [idxv])
        posv = zero16
        new = []
        for e in range(N_EXPERTS):
            m = v == e
            posv = jnp.where(m, lane_base[e] + rs[e], posv)
            new.append(rs[e] + jnp.where(m, 1, 0))
        plsc.store_scatter(pos_v, [idxv], posv)
        return tuple(new)
    lax.fori_loop(0, LPW, pos_body, (zero16,) * N_EXPERTS)
    pltpu.sync_copy(pos_v, pos_hbm.at[pl.ds(base, CHUNK_B)])

    @pl.when(wid == 0)
    def _():
        for j in range(8):
            jv = (jnp.full((16,), j * 16, jnp.int32) + lane) * GTILE
            t = zero16
            for e in range(N_EXPERTS):
                t = t + jnp.where(
                    jv >= jnp.full((16,), end_scal[e], jnp.int32), 1, 0)
            texp_v[pl.ds(j * 16, 16)] = jnp.minimum(t, N_EXPERTS - 1)
        pltpu.sync_copy(texp_v, texp_hbm)


@jax.jit
def _stage_b2(ei, cnths):
    return pl.kernel(
        _route_body,
        mesh=_mesh,
        compiler_params=pltpu.CompilerParams(needs_layout_passes=False),
        out_type=[
            jax.ShapeDtypeStruct((A_TOT,), jnp.int32),
            jax.ShapeDtypeStruct((128,), jnp.int32),
        ],
        scratch_types=[
            pltpu.VMEM((CHUNK_B,), jnp.int32),
            pltpu.VMEM((CHUNK_B,), jnp.int32),
            pltpu.VMEM((NW * 16,), jnp.int32),
            pltpu.VMEM((128,), jnp.int32),
            pltpu.SemaphoreType.DMA,
        ],
    )(ei, cnths)


TOKC = 64


def _disperse_body(h_hbm, pos_hbm, hs_hbm, posc_v, idx0_v, idx1_v,
                   rows_v, sem):
    wid = _wid()
    lane = lax.iota(jnp.int32, 16)

    def body(c, _):
        t0 = wid * TOK_W + c * TOKC
        pltpu.sync_copy(pos_hbm.at[pl.ds(t0 * 2, TOKC * 2)], posc_v)
        pltpu.sync_copy(h_hbm.at[pl.ds(t0, TOKC)], rows_v)
        for j in range(TOKC // 16):
            tl = jnp.full((16,), j * 16, jnp.int32) + lane
            idx0_v[pl.ds(j * 16, 16)] = plsc.load_gather(posc_v, [tl * 2])
            idx1_v[pl.ds(j * 16, 16)] = plsc.load_gather(posc_v,
                                                         [tl * 2 + 1])
        cp0 = pltpu.async_copy(rows_v, hs_hbm.at[idx0_v], sem)
        cp1 = pltpu.async_copy(rows_v, hs_hbm.at[idx1_v], sem)
        cp0.wait()
        cp1.wait()
        return 0
    lax.fori_loop(0, TOK_W // TOKC, body, 0)


@jax.jit
def _stage_c(h_f, pos):
    return pl.kernel(
        _disperse_body,
        mesh=_mesh,
        compiler_params=pltpu.CompilerParams(needs_layout_passes=False),
        out_type=jax.ShapeDtypeStruct((PADTOT, ROWW), jnp.float32),
        scratch_types=[
            pltpu.VMEM((TOKC * 2,), jnp.int32),
            pltpu.VMEM((TOKC,), jnp.int32),
            pltpu.VMEM((TOKC,), jnp.int32),
            pltpu.VMEM((TOKC, ROWW), jnp.float32),
            pltpu.SemaphoreType.DMA,
        ],
    )(h_f, pos)


def _ffn_body(texp_ref, hs_ref, w1_ref, b1_ref, w2_ref, b2_ref, ys_ref):
    hsb = hs_ref[...].astype(jnp.bfloat16)
    hid = lax.dot_general(hsb, w1_ref[0], _NT,
                          preferred_element_type=jnp.float32)
    hid = jnp.maximum(hid + b1_ref[0], 0.0)
    out = lax.dot_general(hid.astype(jnp.bfloat16), w2_ref[0], _NT,
                          preferred_element_type=jnp.float32)
    ys_ref[...] = out + b2_ref[0]


@jax.jit
def _stage_d(texp, hs_bf, W1b, b1, W2b, b2):
    grid_spec = pltpu.PrefetchScalarGridSpec(
        num_scalar_prefetch=1,
        grid=(NTILES,),
        in_specs=[
            pl.BlockSpec((GTILE, HIDDEN), lambda i, t: (i, 0)),
            pl.BlockSpec((1, EXPERT_DIM, HIDDEN), lambda i, t: (t[i], 0, 0)),
            pl.BlockSpec((1, 1, EXPERT_DIM), lambda i, t: (t[i], 0, 0)),
            pl.BlockSpec((1, HIDDEN, EXPERT_DIM), lambda i, t: (t[i], 0, 0)),
            pl.BlockSpec((1, 1, HIDDEN), lambda i, t: (t[i], 0, 0)),
        ],
        out_specs=pl.BlockSpec((GTILE, HIDDEN), lambda i, t: (i, 0)),
    )
    return pl.pallas_call(
        _ffn_body,
        grid_spec=grid_spec,
        out_shape=jax.ShapeDtypeStruct((PADTOT, HIDDEN), jnp.float32),
    )(texp, hs_bf, W1b, b1, W2b, b2)


def _combine_body(ys_hbm, pos_hbm, wi_hbm, h2_hbm, posc_v, wc_v, idx0_v,
                  idx1_v, y0_v, y1_v, out_v, sem):
    wid = _wid()
    lane = lax.iota(jnp.int32, 16)

    def chunk_body(c, _):
        tok0 = wid * TOK_W + c * TCH
        pltpu.sync_copy(pos_hbm.at[pl.ds(tok0 * 2, TCH * 2)], posc_v)
        pltpu.sync_copy(wi_hbm.at[pl.ds(tok0 * 2, TCH * 2)], wc_v)
        for j in range(TCH // 16):
            tl = jnp.full((16,), j * 16, jnp.int32) + lane
            idx0_v[pl.ds(j * 16, 16)] = plsc.load_gather(posc_v, [tl * 2])
            idx1_v[pl.ds(j * 16, 16)] = plsc.load_gather(posc_v, [tl * 2 + 1])
        cp0 = pltpu.async_copy(ys_hbm.at[idx0_v], y0_v, sem)
        cp1 = pltpu.async_copy(ys_hbm.at[idx1_v], y1_v, sem)
        cp0.wait()
        cp1.wait()

        def grp_body(g, _):
            tl = jnp.full((16,), 0, jnp.int32) + lane + g * 16
            w0g = plsc.load_gather(wc_v, [tl * 2])
            w1g = plsc.load_gather(wc_v, [tl * 2 + 1])
            for l in range(16):
                r = g * 16 + l
                w0 = jnp.full((16,), w0g[l], jnp.float32)
                w1 = jnp.full((16,), w1g[l], jnp.float32)
                for cc in range(ROWW // 16):
                    a = y0_v[r, pl.ds(cc * 16, 16)]
                    b = y1_v[r, pl.ds(cc * 16, 16)]
                    out_v[r, pl.ds(cc * 16, 16)] = jnp.maximum(
                        a * w0 + b * w1, 0.0)
            return 0
        lax.fori_loop(0, TCH // 16, grp_body, 0)
        pltpu.sync_copy(out_v, h2_hbm.at[pl.ds(tok0, TCH)])
        return 0
    lax.fori_loop(0, TOK_W // TCH, chunk_body, 0)


@jax.jit
def _stage_e(ys, pos, wi):
    return pl.kernel(
        _combine_body,
        mesh=_mesh,
        compiler_params=pltpu.CompilerParams(needs_layout_passes=False),
        out_type=jax.ShapeDtypeStruct((TOKENS, ROWW), jnp.float32),
        scratch_types=[
            pltpu.VMEM((TCH * 2,), jnp.int32),
            pltpu.VMEM((TCH * 2,), jnp.float32),
            pltpu.VMEM((TCH,), jnp.int32),
            pltpu.VMEM((TCH,), jnp.int32),
            pltpu.VMEM((TCH, ROWW), jnp.float32),
            pltpu.VMEM((TCH, ROWW), jnp.float32),
            pltpu.VMEM((TCH, ROWW), jnp.float32),
            pltpu.SemaphoreType.DMA,
        ],
    )(ys, pos, wi)


def _cls_body(h2_ref, wc_ref, bc_ref, cls_ref):
    cls = lax.dot_general(h2_ref[...].astype(jnp.bfloat16), wc_ref[...], _NT,
                          preferred_element_type=jnp.float32)
    cls_ref[...] = cls + bc_ref[...]


@jax.jit
def _stage_f(h2_bf, Wcb, bc):
    full = lambda *shape: pl.BlockSpec(shape, lambda i: (0,) * len(shape))
    return pl.pallas_call(
        _cls_body,
        grid=(TOKENS // TM,),
        in_specs=[
            pl.BlockSpec((TM, HIDDEN), lambda i: (i, 0)),
            full(N_CLASSES, HIDDEN),
            full(1, N_CLASSES),
        ],
        out_specs=pl.BlockSpec((TM, N_CLASSES), lambda i: (i, 0)),
        out_shape=jax.ShapeDtypeStruct((TOKENS, N_CLASSES), jnp.float32),
    )(h2_bf, Wcb, bc)


def kernel(x, Wp, bp, Wg, W1, b1, W2, b2, Wc, bc):
    Wg_pad = jnp.zeros((EPAD, HIDDEN), jnp.float32).at[:N_EXPERTS].set(Wg)
    probs_pad, h_f, ei_pad, wi_pad = _stage_a(x, Wp, bp[None, :], Wg_pad)
    ei = ei_pad[:, :TOP_K].reshape(-1)
    wi = wi_pad[:, :TOP_K].reshape(-1)
    cnths = _stage_b1(ei)
    pos, texp = _stage_b2(ei, cnths)
    hs = _stage_c(h_f, pos)
    ys = _stage_d(texp, hs, W1.astype(jnp.bfloat16),
                  b1.reshape(N_EXPERTS, 1, EXPERT_DIM),
                  W2.astype(jnp.bfloat16),
                  b2.reshape(N_EXPERTS, 1, HIDDEN))
    h2 = _stage_e(ys, pos, wi)
    cls = _stage_f(h2, Wc.astype(jnp.bfloat16), bc[None, :])
    return cls, probs_pad[:, :N_EXPERTS]


# speedup vs baseline: 4.5787x; 1.2587x over previous
import functools

import jax
import jax.numpy as jnp
from jax import lax
from jax.experimental import pallas as pl
from jax.experimental.pallas import tpu as pltpu
from jax.experimental.pallas import tpu_sc as plsc

TOKENS = 8192
IN_FEATURES = 1024
HIDDEN = 1024
N_CLASSES = 1000
N_EXPERTS = 8
TOP_K = 2
EXPERT_DIM = 256

TM = 512
EPAD = 128
A_TOT = TOKENS * TOP_K
GTILE = 256
PADTOT = A_TOT + N_EXPERTS * GTILE
NTILES = PADTOT // GTILE
NW = 32
CHUNK_B = A_TOT // NW
ZCHUNK = PADTOT // NW
ROWW = HIDDEN
GCH = 64
TCH = 32
TOK_W = TOKENS // NW

_NT = (((1,), (1,)), ((), ()))

_mesh = plsc.VectorSubcoreMesh(core_axis_name="c", subcore_axis_name="s")


def _wid():
    return lax.axis_index("s") * 2 + lax.axis_index("c")


def _proj_router_body(x_ref, wp_ref, bp_ref, wg_ref,
                      probs_ref, h_ref, ei_ref, wi_ref):
    x = x_ref[...]
    h = lax.dot_general(x, wp_ref[...], _NT, preferred_element_type=jnp.float32)
    h = jnp.maximum(h + bp_ref[...], 0.0)
    h_ref[...] = h

    logits = lax.dot_general(h, wg_ref[...], _NT,
                             preferred_element_type=jnp.float32)
    col = lax.broadcasted_iota(jnp.int32, (TM, EPAD), 1)
    logits = jnp.where(col < N_EXPERTS, logits, jnp.float32(-1e30))
    lmax = jnp.max(logits, axis=1, keepdims=True)
    ex = jnp.exp(logits - lmax)
    probs = ex / jnp.sum(ex, axis=1, keepdims=True)
    probs_ref[...] = probs

    w1 = jnp.max(probs, axis=1, keepdims=True)
    i1 = jnp.min(jnp.where(probs == w1, col, EPAD), axis=1, keepdims=True)
    probs2 = jnp.where(col == i1, -1.0, probs)
    w2 = jnp.max(probs2, axis=1, keepdims=True)
    i2 = jnp.min(jnp.where(probs2 == w2, col, EPAD), axis=1, keepdims=True)
    s = w1 + w2
    ei_ref[...] = jnp.where(col == 0, i1, jnp.where(col == 1, i2, 0))
    wi_ref[...] = jnp.where(col == 0, w1 / s, jnp.where(col == 1, w2 / s, 0.0))


@jax.jit
def _stage_a(x, Wp, bp, Wg_pad):
    full = lambda *shape: pl.BlockSpec(shape, lambda i: (0,) * len(shape))
    return pl.pallas_call(
        _proj_router_body,
        grid=(TOKENS // TM,),
        in_specs=[
            pl.BlockSpec((TM, IN_FEATURES), lambda i: (i, 0)),
            full(HIDDEN, IN_FEATURES),
            full(1, HIDDEN),
            full(EPAD, HIDDEN),
        ],
        out_specs=[
            pl.BlockSpec((TM, EPAD), lambda i: (i, 0)),
            pl.BlockSpec((TM, HIDDEN), lambda i: (i, 0)),
            pl.BlockSpec((TM, EPAD), lambda i: (i, 0)),
            pl.BlockSpec((TM, EPAD), lambda i: (i, 0)),
        ],
        out_shape=[
            jax.ShapeDtypeStruct((TOKENS, EPAD), jnp.float32),
            jax.ShapeDtypeStruct((TOKENS, HIDDEN), jnp.float32),
            jax.ShapeDtypeStruct((TOKENS, EPAD), jnp.int32),
            jax.ShapeDtypeStruct((TOKENS, EPAD), jnp.float32),
        ],
    )(x, Wp, bp, Wg_pad)


LPW = CHUNK_B // 16


def _hist(ids_v, lane, zero16):
    def hist_body(i, accs):
        v = plsc.load_gather(ids_v, [lane * LPW + i])
        return tuple(accs[e] + jnp.where(v == e, 1, 0)
                     for e in range(N_EXPERTS))
    return lax.fori_loop(0, LPW, hist_body, (zero16,) * N_EXPERTS)


def _hist_body_k(ei_hbm, cnths_hbm, ids_v, cnt_v, sem):
    wid = _wid()
    base = wid * CHUNK_B
    lane = lax.iota(jnp.int32, 16)
    zero16 = jnp.zeros((16,), jnp.int32)
    pltpu.sync_copy(ei_hbm.at[pl.ds(base, CHUNK_B)], ids_v)
    accs = _hist(ids_v, lane, zero16)
    cnt = zero16
    for e in range(N_EXPERTS):
        acc = accs[e]
        tot_e = acc[0]
        for l in range(1, 16):
            tot_e = tot_e + acc[l]
        cnt = jnp.where(lane == e, jnp.full((16,), tot_e, jnp.int32), cnt)
    cnt_v[...] = cnt
    pltpu.sync_copy(cnt_v, cnths_hbm.at[pl.ds(wid * 16, 16)])


@jax.jit
def _stage_b1(ei):
    return pl.kernel(
        _hist_body_k,
        mesh=_mesh,
        compiler_params=pltpu.CompilerParams(needs_layout_passes=False),
        out_type=jax.ShapeDtypeStruct((NW * 16,), jnp.int32),
        scratch_types=[
            pltpu.VMEM((CHUNK_B,), jnp.int32),
            pltpu.VMEM((16,), jnp.int32),
            pltpu.SemaphoreType.DMA,
        ],
    )(ei)


def _route_body(ei_hbm, cnths_hbm, pos_hbm, texp_hbm,
                ids_v, pos_v, allcnt_v, texp_v, sem):
    wid = _wid()
    base = wid * CHUNK_B
    lane = lax.iota(jnp.int32, 16)
    zero16 = jnp.zeros((16,), jnp.int32)

    pltpu.sync_copy(ei_hbm.at[pl.ds(base, CHUNK_B)], ids_v)
    pltpu.sync_copy(cnths_hbm, allcnt_v)
    accs = _hist(ids_v, lane, zero16)

    widv = jnp.full((16,), wid, jnp.int32)
    tot = zero16
    mybase_cnt = zero16
    for w in range(NW):
        row = allcnt_v[pl.ds(w * 16, 16)]
        tot = tot + row
        mybase_cnt = mybase_cnt + jnp.where(
            jnp.full((16,), w, jnp.int32) < widv, row, 0)

    end_scal = []
    lane_base = []
    gs_run = jnp.int32(0)
    for e in range(N_EXPERTS):
        tot_e = tot[e]
        pcnt_e = jnp.bitwise_and(tot_e + (GTILE - 1), ~(GTILE - 1))
        base_e = gs_run + mybase_cnt[e]
        gs_run = gs_run + pcnt_e
        end_scal.append(gs_run)
        vec = zero16
        run_s = base_e
        acc = accs[e]
        for l in range(16):
            vec = jnp.where(lane == l, jnp.full((16,), run_s, jnp.int32), vec)
            run_s = run_s + acc[l]
        lane_base.append(vec)

    def pos_body(i, rs):
        idxv = lane * LPW + i
        v = plsc.load_gather(ids_v, [idxv])
        posv = zero16
        new = []
        for e in range(N_EXPERTS):
            m = v == e
            posv = jnp.where(m, lane_base[e] + rs[e], posv)
            new.append(rs[e] + jnp.where(m, 1, 0))
        plsc.store_scatter(pos_v, [idxv], posv)
        return tuple(new)
    lax.fori_loop(0, LPW, pos_body, (zero16,) * N_EXPERTS)
    pltpu.sync_copy(pos_v, pos_hbm.at[pl.ds(base, CHUNK_B)])

    @pl.when(wid == 0)
    def _():
        for j in range(8):
            jv = (jnp.full((16,), j * 16, jnp.int32) + lane) * GTILE
            t = zero16
            for e in range(N_EXPERTS):
                t = t + jnp.where(
                    jv >= jnp.full((16,), end_scal[e], jnp.int32), 1, 0)
            texp_v[pl.ds(j * 16, 16)] = jnp.minimum(t, N_EXPERTS - 1)
        pltpu.sync_copy(texp_v, texp_hbm)


@jax.jit
def _stage_b2(ei, cnths):
    return pl.kernel(
        _route_body,
        mesh=_mesh,
        compiler_params=pltpu.CompilerParams(needs_layout_passes=False),
        out_type=[
            jax.ShapeDtypeStruct((A_TOT,), jnp.int32),
            jax.ShapeDtypeStruct((128,), jnp.int32),
        ],
        scratch_types=[
            pltpu.VMEM((CHUNK_B,), jnp.int32),
            pltpu.VMEM((CHUNK_B,), jnp.int32),
            pltpu.VMEM((NW * 16,), jnp.int32),
            pltpu.VMEM((128,), jnp.int32),
            pltpu.SemaphoreType.DMA,
        ],
    )(ei, cnths)


TOKC = 64


def _disperse_body(h_hbm, pos_hbm, hs_hbm, posc_v, idx0_v, idx1_v,
                   rows_v, sem):
    wid = _wid()
    lane = lax.iota(jnp.int32, 16)

    def body(c, _):
        t0 = wid * TOK_W + c * TOKC
        pltpu.sync_copy(pos_hbm.at[pl.ds(t0 * 2, TOKC * 2)], posc_v)
        pltpu.sync_copy(h_hbm.at[pl.ds(t0, TOKC)], rows_v)
        for j in range(TOKC // 16):
            tl = jnp.full((16,), j * 16, jnp.int32) + lane
            idx0_v[pl.ds(j * 16, 16)] = plsc.load_gather(posc_v, [tl * 2])
            idx1_v[pl.ds(j * 16, 16)] = plsc.load_gather(posc_v,
                                                         [tl * 2 + 1])
        cp0 = pltpu.async_copy(rows_v, hs_hbm.at[idx0_v], sem)
        cp1 = pltpu.async_copy(rows_v, hs_hbm.at[idx1_v], sem)
        cp0.wait()
        cp1.wait()
        return 0
    lax.fori_loop(0, TOK_W // TOKC, body, 0)


@jax.jit
def _stage_c(h_f, pos):
    return pl.kernel(
        _disperse_body,
        mesh=_mesh,
        compiler_params=pltpu.CompilerParams(needs_layout_passes=False),
        out_type=jax.ShapeDtypeStruct((PADTOT, ROWW), jnp.float32),
        scratch_types=[
            pltpu.VMEM((TOKC * 2,), jnp.int32),
            pltpu.VMEM((TOKC,), jnp.int32),
            pltpu.VMEM((TOKC,), jnp.int32),
            pltpu.VMEM((TOKC, ROWW), jnp.float32),
            pltpu.SemaphoreType.DMA,
        ],
    )(h_f, pos)


def _ffn_body(texp_ref, hs_ref, w1_ref, b1_ref, w2_ref, b2_ref, ys_ref):
    hsb = hs_ref[...].astype(jnp.bfloat16)
    hid = lax.dot_general(hsb, w1_ref[0], _NT,
                          preferred_element_type=jnp.float32)
    hid = jnp.maximum(hid + b1_ref[0], 0.0)
    out = lax.dot_general(hid.astype(jnp.bfloat16), w2_ref[0], _NT,
                          preferred_element_type=jnp.float32)
    ys_ref[...] = out + b2_ref[0]


@jax.jit
def _stage_d(texp, hs_bf, W1b, b1, W2b, b2):
    grid_spec = pltpu.PrefetchScalarGridSpec(
        num_scalar_prefetch=1,
        grid=(NTILES,),
        in_specs=[
            pl.BlockSpec((GTILE, HIDDEN), lambda i, t: (i, 0)),
            pl.BlockSpec((1, EXPERT_DIM, HIDDEN), lambda i, t: (t[i], 0, 0)),
            pl.BlockSpec((1, 1, EXPERT_DIM), lambda i, t: (t[i], 0, 0)),
            pl.BlockSpec((1, HIDDEN, EXPERT_DIM), lambda i, t: (t[i], 0, 0)),
            pl.BlockSpec((1, 1, HIDDEN), lambda i, t: (t[i], 0, 0)),
        ],
        out_specs=pl.BlockSpec((GTILE, HIDDEN), lambda i, t: (i, 0)),
    )
    return pl.pallas_call(
        _ffn_body,
        grid_spec=grid_spec,
        out_shape=jax.ShapeDtypeStruct((PADTOT, HIDDEN), jnp.float32),
    )(texp, hs_bf, W1b, b1, W2b, b2)


def _combine_body(ys_hbm, pos_hbm, y0_hbm, y1_hbm, posc_v, idx0_v,
                  idx1_v, y0_v, y1_v, sem):
    wid = _wid()
    lane = lax.iota(jnp.int32, 16)

    def chunk_body(c, _):
        tok0 = wid * TOK_W + c * TCH
        pltpu.sync_copy(pos_hbm.at[pl.ds(tok0 * 2, TCH * 2)], posc_v)
        for j in range(TCH // 16):
            tl = jnp.full((16,), j * 16, jnp.int32) + lane
            idx0_v[pl.ds(j * 16, 16)] = plsc.load_gather(posc_v, [tl * 2])
            idx1_v[pl.ds(j * 16, 16)] = plsc.load_gather(posc_v, [tl * 2 + 1])
        cp0 = pltpu.async_copy(ys_hbm.at[idx0_v], y0_v, sem)
        cp1 = pltpu.async_copy(ys_hbm.at[idx1_v], y1_v, sem)
        cp0.wait()
        cp1.wait()
        cp2 = pltpu.async_copy(y0_v, y0_hbm.at[pl.ds(tok0, TCH)], sem)
        cp3 = pltpu.async_copy(y1_v, y1_hbm.at[pl.ds(tok0, TCH)], sem)
        cp2.wait()
        cp3.wait()
        return 0
    lax.fori_loop(0, TOK_W // TCH, chunk_body, 0)


@jax.jit
def _stage_e(ys, pos):
    return pl.kernel(
        _combine_body,
        mesh=_mesh,
        compiler_params=pltpu.CompilerParams(needs_layout_passes=False),
        out_type=[
            jax.ShapeDtypeStruct((TOKENS, ROWW), jnp.float32),
            jax.ShapeDtypeStruct((TOKENS, ROWW), jnp.float32),
        ],
        scratch_types=[
            pltpu.VMEM((TCH * 2,), jnp.int32),
            pltpu.VMEM((TCH,), jnp.int32),
            pltpu.VMEM((TCH,), jnp.int32),
            pltpu.VMEM((TCH, ROWW), jnp.float32),
            pltpu.VMEM((TCH, ROWW), jnp.float32),
            pltpu.SemaphoreType.DMA,
        ],
    )(ys, pos)


def _cls_body(y0_ref, y1_ref, wp_ref, wc_ref, bc_ref, cls_ref):
    w0 = wp_ref[:, 0:1]
    w1 = wp_ref[:, 1:2]
    h2 = jnp.maximum(y0_ref[...] * w0 + y1_ref[...] * w1, 0.0)
    cls = lax.dot_general(h2.astype(jnp.bfloat16), wc_ref[...], _NT,
                          preferred_element_type=jnp.float32)
    cls_ref[...] = cls + bc_ref[...]


@jax.jit
def _stage_f(y0, y1, wi_pad, Wcb, bc):
    full = lambda *shape: pl.BlockSpec(shape, lambda i: (0,) * len(shape))
    return pl.pallas_call(
        _cls_body,
        grid=(TOKENS // TM,),
        in_specs=[
            pl.BlockSpec((TM, HIDDEN), lambda i: (i, 0)),
            pl.BlockSpec((TM, HIDDEN), lambda i: (i, 0)),
            pl.BlockSpec((TM, EPAD), lambda i: (i, 0)),
            full(N_CLASSES, HIDDEN),
            full(1, N_CLASSES),
        ],
        out_specs=pl.BlockSpec((TM, N_CLASSES), lambda i: (i, 0)),
        out_shape=jax.ShapeDtypeStruct((TOKENS, N_CLASSES), jnp.float32),
    )(y0, y1, wi_pad, Wcb, bc)


def kernel(x, Wp, bp, Wg, W1, b1, W2, b2, Wc, bc):
    Wg_pad = jnp.zeros((EPAD, HIDDEN), jnp.float32).at[:N_EXPERTS].set(Wg)
    probs_pad, h_f, ei_pad, wi_pad = _stage_a(x, Wp, bp[None, :], Wg_pad)
    ei = ei_pad[:, :TOP_K].reshape(-1)
    wi = wi_pad[:, :TOP_K].reshape(-1)
    cnths = _stage_b1(ei)
    pos, texp = _stage_b2(ei, cnths)
    hs = _stage_c(h_f, pos)
    ys = _stage_d(texp, hs, W1.astype(jnp.bfloat16),
                  b1.reshape(N_EXPERTS, 1, EXPERT_DIM),
                  W2.astype(jnp.bfloat16),
                  b2.reshape(N_EXPERTS, 1, HIDDEN))
    y0, y1 = _stage_e(ys, pos)
    cls = _stage_f(y0, y1, wi_pad, Wc.astype(jnp.bfloat16), bc[None, :])
    return cls, probs_pad[:, :N_EXPERTS]


# speedup vs baseline: 4.6550x; 1.0167x over previous
import functools

import jax
import jax.numpy as jnp
from jax import lax
from jax.experimental import pallas as pl
from jax.experimental.pallas import tpu as pltpu
from jax.experimental.pallas import tpu_sc as plsc

TOKENS = 8192
IN_FEATURES = 1024
HIDDEN = 1024
N_CLASSES = 1000
N_EXPERTS = 8
TOP_K = 2
EXPERT_DIM = 256

TM = 512
EPAD = 128
A_TOT = TOKENS * TOP_K
GTILE = 256
PADTOT = A_TOT + N_EXPERTS * GTILE
NTILES = PADTOT // GTILE
NW = 32
CHUNK_B = A_TOT // NW
ZCHUNK = PADTOT // NW
ROWW = HIDDEN
GCH = 64
TCH = 32
TOKC = 64
TOK_W = TOKENS // NW

_NT = (((1,), (1,)), ((), ()))

_mesh = plsc.VectorSubcoreMesh(core_axis_name="c", subcore_axis_name="s")


def _wid():
    return lax.axis_index("s") * 2 + lax.axis_index("c")


def _proj_router_body(x_ref, wp_ref, bp_ref, wg_ref,
                      probs_ref, h_ref, ei_ref, wi_ref):
    x = x_ref[...]
    h = lax.dot_general(x, wp_ref[...], _NT, preferred_element_type=jnp.float32)
    h = jnp.maximum(h + bp_ref[...], 0.0)
    h_ref[...] = h

    logits = lax.dot_general(h, wg_ref[...], _NT,
                             preferred_element_type=jnp.float32)
    col = lax.broadcasted_iota(jnp.int32, (TM, EPAD), 1)
    logits = jnp.where(col < N_EXPERTS, logits, jnp.float32(-1e30))
    lmax = jnp.max(logits, axis=1, keepdims=True)
    ex = jnp.exp(logits - lmax)
    probs = ex / jnp.sum(ex, axis=1, keepdims=True)
    probs_ref[...] = probs

    w1 = jnp.max(probs, axis=1, keepdims=True)
    i1 = jnp.min(jnp.where(probs == w1, col, EPAD), axis=1, keepdims=True)
    probs2 = jnp.where(col == i1, -1.0, probs)
    w2 = jnp.max(probs2, axis=1, keepdims=True)
    i2 = jnp.min(jnp.where(probs2 == w2, col, EPAD), axis=1, keepdims=True)
    s = w1 + w2
    ei_ref[...] = jnp.where(col == 0, i1, jnp.where(col == 1, i2, 0))
    wi_ref[...] = jnp.where(col == 0, w1 / s, jnp.where(col == 1, w2 / s, 0.0))


@jax.jit
def _stage_a(x, Wp, bp, Wg_pad):
    full = lambda *shape: pl.BlockSpec(shape, lambda i: (0,) * len(shape))
    return pl.pallas_call(
        _proj_router_body,
        grid=(TOKENS // TM,),
        in_specs=[
            pl.BlockSpec((TM, IN_FEATURES), lambda i: (i, 0)),
            full(HIDDEN, IN_FEATURES),
            full(1, HIDDEN),
            full(EPAD, HIDDEN),
        ],
        out_specs=[
            pl.BlockSpec((TM, EPAD), lambda i: (i, 0)),
            pl.BlockSpec((TM, HIDDEN), lambda i: (i, 0)),
            pl.BlockSpec((TM, EPAD), lambda i: (i, 0)),
            pl.BlockSpec((TM, EPAD), lambda i: (i, 0)),
        ],
        out_shape=[
            jax.ShapeDtypeStruct((TOKENS, EPAD), jnp.float32),
            jax.ShapeDtypeStruct((TOKENS, HIDDEN), jnp.float32),
            jax.ShapeDtypeStruct((TOKENS, EPAD), jnp.int32),
            jax.ShapeDtypeStruct((TOKENS, EPAD), jnp.float32),
        ],
    )(x, Wp, bp, Wg_pad)


LPW = CHUNK_B // 16


def _hist(ids_v, lane, zero16):
    def hist_body(i, accs):
        v = plsc.load_gather(ids_v, [lane * LPW + i])
        return tuple(accs[e] + jnp.where(v == e, 1, 0)
                     for e in range(N_EXPERTS))
    return lax.fori_loop(0, LPW, hist_body, (zero16,) * N_EXPERTS)


def _hist_body_k(ei_hbm, cnths_hbm, ids_v, cnt_v, sem):
    wid = _wid()
    base = wid * CHUNK_B
    lane = lax.iota(jnp.int32, 16)
    zero16 = jnp.zeros((16,), jnp.int32)
    pltpu.sync_copy(ei_hbm.at[pl.ds(base, CHUNK_B)], ids_v)
    accs = _hist(ids_v, lane, zero16)
    cnt = zero16
    for e in range(N_EXPERTS):
        acc = accs[e]
        tot_e = acc[0]
        for l in range(1, 16):
            tot_e = tot_e + acc[l]
        cnt = jnp.where(lane == e, jnp.full((16,), tot_e, jnp.int32), cnt)
    cnt_v[...] = cnt
    pltpu.sync_copy(cnt_v, cnths_hbm.at[pl.ds(wid * 16, 16)])


@jax.jit
def _stage_b1(ei):
    return pl.kernel(
        _hist_body_k,
        mesh=_mesh,
        compiler_params=pltpu.CompilerParams(needs_layout_passes=False),
        out_type=jax.ShapeDtypeStruct((NW * 16,), jnp.int32),
        scratch_types=[
            pltpu.VMEM((CHUNK_B,), jnp.int32),
            pltpu.VMEM((16,), jnp.int32),
            pltpu.SemaphoreType.DMA,
        ],
    )(ei)


def _route_body(ei_hbm, cnths_hbm, h_hbm, pos_hbm, texp_hbm, hs_hbm,
                ids_v, pos_v, allcnt_v, texp_v, idx0_v, idx1_v, rows_v,
                sem):
    wid = _wid()
    base = wid * CHUNK_B
    lane = lax.iota(jnp.int32, 16)
    zero16 = jnp.zeros((16,), jnp.int32)

    pltpu.sync_copy(ei_hbm.at[pl.ds(base, CHUNK_B)], ids_v)
    pltpu.sync_copy(cnths_hbm, allcnt_v)
    accs = _hist(ids_v, lane, zero16)

    widv = jnp.full((16,), wid, jnp.int32)
    tot = zero16
    mybase_cnt = zero16
    for w in range(NW):
        row = allcnt_v[pl.ds(w * 16, 16)]
        tot = tot + row
        mybase_cnt = mybase_cnt + jnp.where(
            jnp.full((16,), w, jnp.int32) < widv, row, 0)

    end_scal = []
    lane_base = []
    gs_run = jnp.int32(0)
    for e in range(N_EXPERTS):
        tot_e = tot[e]
        pcnt_e = jnp.bitwise_and(tot_e + (GTILE - 1), ~(GTILE - 1))
        base_e = gs_run + mybase_cnt[e]
        gs_run = gs_run + pcnt_e
        end_scal.append(gs_run)
        vec = zero16
        run_s = base_e
        acc = accs[e]
        for l in range(16):
            vec = jnp.where(lane == l, jnp.full((16,), run_s, jnp.int32), vec)
            run_s = run_s + acc[l]
        lane_base.append(vec)

    def pos_body(i, rs):
        idxv = lane * LPW + i
        v = plsc.load_gather(ids_v, [idxv])
        posv = zero16
        new = []
        for e in range(N_EXPERTS):
            m = v == e
            posv = jnp.where(m, lane_base[e] + rs[e], posv)
            new.append(rs[e] + jnp.where(m, 1, 0))
        plsc.store_scatter(pos_v, [idxv], posv)
        return tuple(new)
    lax.fori_loop(0, LPW, pos_body, (zero16,) * N_EXPERTS)
    pltpu.sync_copy(pos_v, pos_hbm.at[pl.ds(base, CHUNK_B)])

    def dis_body(c, _):
        t0 = wid * TOK_W + c * TOKC
        a0 = c * TOKC * 2
        pltpu.sync_copy(h_hbm.at[pl.ds(t0, TOKC)], rows_v)
        for j in range(TOKC // 16):
            tl = jnp.full((16,), a0 + j * 32, jnp.int32) + lane * 2
            idx0_v[pl.ds(j * 16, 16)] = plsc.load_gather(pos_v, [tl])
            idx1_v[pl.ds(j * 16, 16)] = plsc.load_gather(pos_v, [tl + 1])
        cp0 = pltpu.async_copy(rows_v, hs_hbm.at[idx0_v], sem)
        cp1 = pltpu.async_copy(rows_v, hs_hbm.at[idx1_v], sem)
        cp0.wait()
        cp1.wait()
        return 0
    lax.fori_loop(0, TOK_W // TOKC, dis_body, 0)

    @pl.when(wid == 0)
    def _():
        for j in range(8):
            jv = (jnp.full((16,), j * 16, jnp.int32) + lane) * GTILE
            t = zero16
            for e in range(N_EXPERTS):
                t = t + jnp.where(
                    jv >= jnp.full((16,), end_scal[e], jnp.int32), 1, 0)
            texp_v[pl.ds(j * 16, 16)] = jnp.minimum(t, N_EXPERTS - 1)
        pltpu.sync_copy(texp_v, texp_hbm)


@jax.jit
def _stage_b2(ei, cnths, h_f):
    return pl.kernel(
        _route_body,
        mesh=_mesh,
        compiler_params=pltpu.CompilerParams(needs_layout_passes=False),
        out_type=[
            jax.ShapeDtypeStruct((A_TOT,), jnp.int32),
            jax.ShapeDtypeStruct((128,), jnp.int32),
            jax.ShapeDtypeStruct((PADTOT, ROWW), jnp.float32),
        ],
        scratch_types=[
            pltpu.VMEM((CHUNK_B,), jnp.int32),
            pltpu.VMEM((CHUNK_B,), jnp.int32),
            pltpu.VMEM((NW * 16,), jnp.int32),
            pltpu.VMEM((128,), jnp.int32),
            pltpu.VMEM((TOKC,), jnp.int32),
            pltpu.VMEM((TOKC,), jnp.int32),
            pltpu.VMEM((TOKC, ROWW), jnp.float32),
            pltpu.SemaphoreType.DMA,
        ],
    )(ei, cnths, h_f)


def _ffn_body(texp_ref, hs_ref, w1_ref, b1_ref, w2_ref, b2_ref, ys_ref):
    hsb = hs_ref[...].astype(jnp.bfloat16)
    hid = lax.dot_general(hsb, w1_ref[0], _NT,
                          preferred_element_type=jnp.float32)
    hid = jnp.maximum(hid + b1_ref[0], 0.0)
    out = lax.dot_general(hid.astype(jnp.bfloat16), w2_ref[0], _NT,
                          preferred_element_type=jnp.float32)
    ys_ref[...] = out + b2_ref[0]


@jax.jit
def _stage_d(texp, hs_bf, W1b, b1, W2b, b2):
    grid_spec = pltpu.PrefetchScalarGridSpec(
        num_scalar_prefetch=1,
        grid=(NTILES,),
        in_specs=[
            pl.BlockSpec((GTILE, HIDDEN), lambda i, t: (i, 0)),
            pl.BlockSpec((1, EXPERT_DIM, HIDDEN), lambda i, t: (t[i], 0, 0)),
            pl.BlockSpec((1, 1, EXPERT_DIM), lambda i, t: (t[i], 0, 0)),
            pl.BlockSpec((1, HIDDEN, EXPERT_DIM), lambda i, t: (t[i], 0, 0)),
            pl.BlockSpec((1, 1, HIDDEN), lambda i, t: (t[i], 0, 0)),
        ],
        out_specs=pl.BlockSpec((GTILE, HIDDEN), lambda i, t: (i, 0)),
    )
    return pl.pallas_call(
        _ffn_body,
        grid_spec=grid_spec,
        out_shape=jax.ShapeDtypeStruct((PADTOT, HIDDEN), jnp.float32),
    )(texp, hs_bf, W1b, b1, W2b, b2)


def _combine_body(ys_hbm, pos_hbm, y0_hbm, y1_hbm, posc_v, idx0_v,
                  idx1_v, y0_v, y1_v, sem):
    wid = _wid()
    lane = lax.iota(jnp.int32, 16)

    def chunk_body(c, _):
        tok0 = wid * TOK_W + c * TCH
        pltpu.sync_copy(pos_hbm.at[pl.ds(tok0 * 2, TCH * 2)], posc_v)
        for j in range(TCH // 16):
            tl = jnp.full((16,), j * 16, jnp.int32) + lane
            idx0_v[pl.ds(j * 16, 16)] = plsc.load_gather(posc_v, [tl * 2])
            idx1_v[pl.ds(j * 16, 16)] = plsc.load_gather(posc_v, [tl * 2 + 1])
        cp0 = pltpu.async_copy(ys_hbm.at[idx0_v], y0_v, sem)
        cp1 = pltpu.async_copy(ys_hbm.at[idx1_v], y1_v, sem)
        cp0.wait()
        cp1.wait()
        cp2 = pltpu.async_copy(y0_v, y0_hbm.at[pl.ds(tok0, TCH)], sem)
        cp3 = pltpu.async_copy(y1_v, y1_hbm.at[pl.ds(tok0, TCH)], sem)
        cp2.wait()
        cp3.wait()
        return 0
    lax.fori_loop(0, TOK_W // TCH, chunk_body, 0)


@jax.jit
def _stage_e(ys, pos):
    return pl.kernel(
        _combine_body,
        mesh=_mesh,
        compiler_params=pltpu.CompilerParams(needs_layout_passes=False),
        out_type=[
            jax.ShapeDtypeStruct((TOKENS, ROWW), jnp.float32),
            jax.ShapeDtypeStruct((TOKENS, ROWW), jnp.float32),
        ],
        scratch_types=[
            pltpu.VMEM((TCH * 2,), jnp.int32),
            pltpu.VMEM((TCH,), jnp.int32),
            pltpu.VMEM((TCH,), jnp.int32),
            pltpu.VMEM((TCH, ROWW), jnp.float32),
            pltpu.VMEM((TCH, ROWW), jnp.float32),
            pltpu.SemaphoreType.DMA,
        ],
    )(ys, pos)


def _cls_body(y0_ref, y1_ref, wp_ref, wc_ref, bc_ref, cls_ref):
    w0 = wp_ref[:, 0:1]
    w1 = wp_ref[:, 1:2]
    h2 = jnp.maximum(y0_ref[...] * w0 + y1_ref[...] * w1, 0.0)
    cls = lax.dot_general(h2.astype(jnp.bfloat16), wc_ref[...], _NT,
                          preferred_element_type=jnp.float32)
    cls_ref[...] = cls + bc_ref[...]


@jax.jit
def _stage_f(y0, y1, wi_pad, Wcb, bc):
    full = lambda *shape: pl.BlockSpec(shape, lambda i: (0,) * len(shape))
    return pl.pallas_call(
        _cls_body,
        grid=(TOKENS // TM,),
        in_specs=[
            pl.BlockSpec((TM, HIDDEN), lambda i: (i, 0)),
            pl.BlockSpec((TM, HIDDEN), lambda i: (i, 0)),
            pl.BlockSpec((TM, EPAD), lambda i: (i, 0)),
            full(N_CLASSES, HIDDEN),
            full(1, N_CLASSES),
        ],
        out_specs=pl.BlockSpec((TM, N_CLASSES), lambda i: (i, 0)),
        out_shape=jax.ShapeDtypeStruct((TOKENS, N_CLASSES), jnp.float32),
    )(y0, y1, wi_pad, Wcb, bc)


def kernel(x, Wp, bp, Wg, W1, b1, W2, b2, Wc, bc):
    Wg_pad = jnp.zeros((EPAD, HIDDEN), jnp.float32).at[:N_EXPERTS].set(Wg)
    probs_pad, h_f, ei_pad, wi_pad = _stage_a(x, Wp, bp[None, :], Wg_pad)
    ei = ei_pad[:, :TOP_K].reshape(-1)
    wi = wi_pad[:, :TOP_K].reshape(-1)
    cnths = _stage_b1(ei)
    pos, texp, hs = _stage_b2(ei, cnths, h_f)
    ys = _stage_d(texp, hs, W1.astype(jnp.bfloat16),
                  b1.reshape(N_EXPERTS, 1, EXPERT_DIM),
                  W2.astype(jnp.bfloat16),
                  b2.reshape(N_EXPERTS, 1, HIDDEN))
    y0, y1 = _stage_e(ys, pos)
    cls = _stage_f(y0, y1, wi_pad, Wc.astype(jnp.bfloat16), bc[None, :])
    return cls, probs_pad[:, :N_EXPERTS]


# speedup vs baseline: 4.7247x; 1.0150x over previous
import functools

import jax
import jax.numpy as jnp
from jax import lax
from jax.experimental import pallas as pl
from jax.experimental.pallas import tpu as pltpu
from jax.experimental.pallas import tpu_sc as plsc

TOKENS = 8192
IN_FEATURES = 1024
HIDDEN = 1024
N_CLASSES = 1000
N_EXPERTS = 8
TOP_K = 2
EXPERT_DIM = 256

TM = 512
EPAD = 128
A_TOT = TOKENS * TOP_K
GTILE = 256
PADTOT = A_TOT + N_EXPERTS * GTILE
NTILES = PADTOT // GTILE
NW = 32
CHUNK_B = A_TOT // NW
ZCHUNK = PADTOT // NW
ROWW = HIDDEN
GCH = 64
TCH = 32
TOKC = 64
TOK_W = TOKENS // NW

_NT = (((1,), (1,)), ((), ()))

_mesh = plsc.VectorSubcoreMesh(core_axis_name="c", subcore_axis_name="s")


def _wid():
    return lax.axis_index("s") * 2 + lax.axis_index("c")


def _proj_router_body(x_ref, wp_ref, bp_ref, wg_ref,
                      probs_ref, h_ref, ei_ref, wi_ref, cnt_ref):
    x = x_ref[...]
    h = lax.dot_general(x, wp_ref[...], _NT, preferred_element_type=jnp.float32)
    h = jnp.maximum(h + bp_ref[...], 0.0)
    h_ref[...] = h

    logits = lax.dot_general(h, wg_ref[...], _NT,
                             preferred_element_type=jnp.float32)
    col = lax.broadcasted_iota(jnp.int32, (TM, EPAD), 1)
    logits = jnp.where(col < N_EXPERTS, logits, jnp.float32(-1e30))
    lmax = jnp.max(logits, axis=1, keepdims=True)
    ex = jnp.exp(logits - lmax)
    probs = ex / jnp.sum(ex, axis=1, keepdims=True)
    probs_ref[...] = probs

    w1 = jnp.max(probs, axis=1, keepdims=True)
    i1 = jnp.min(jnp.where(probs == w1, col, EPAD), axis=1, keepdims=True)
    probs2 = jnp.where(col == i1, -1.0, probs)
    w2 = jnp.max(probs2, axis=1, keepdims=True)
    i2 = jnp.min(jnp.where(probs2 == w2, col, EPAD), axis=1, keepdims=True)
    s = w1 + w2
    ei_ref[...] = jnp.where(col == 0, i1, jnp.where(col == 1, i2, 0))
    wi_ref[...] = jnp.where(col == 0, w1 / s, jnp.where(col == 1, w2 / s, 0.0))
    oh = (jnp.where(col == i1, 1.0, 0.0) + jnp.where(col == i2, 1.0, 0.0))
    c0 = jnp.sum(oh[:TM // 2], axis=0, keepdims=True)
    c1 = jnp.sum(oh[TM // 2:], axis=0, keepdims=True)
    cnt_ref[...] = jnp.concatenate([c0, c1], axis=0).astype(jnp.int32)[None]


@jax.jit
def _stage_a(x, Wp, bp, Wg_pad):
    full = lambda *shape: pl.BlockSpec(shape, lambda i: (0,) * len(shape))
    return pl.pallas_call(
        _proj_router_body,
        grid=(TOKENS // TM,),
        in_specs=[
            pl.BlockSpec((TM, IN_FEATURES), lambda i: (i, 0)),
            full(HIDDEN, IN_FEATURES),
            full(1, HIDDEN),
            full(EPAD, HIDDEN),
        ],
        out_specs=[
            pl.BlockSpec((TM, EPAD), lambda i: (i, 0)),
            pl.BlockSpec((TM, HIDDEN), lambda i: (i, 0)),
            pl.BlockSpec((TM, EPAD), lambda i: (i, 0)),
            pl.BlockSpec((TM, EPAD), lambda i: (i, 0)),
            pl.BlockSpec((1, 2, EPAD), lambda i: (i, 0, 0)),
        ],
        out_shape=[
            jax.ShapeDtypeStruct((TOKENS, EPAD), jnp.float32),
            jax.ShapeDtypeStruct((TOKENS, HIDDEN), jnp.float32),
            jax.ShapeDtypeStruct((TOKENS, EPAD), jnp.int32),
            jax.ShapeDtypeStruct((TOKENS, EPAD), jnp.float32),
            jax.ShapeDtypeStruct((NW // 2, 2, EPAD), jnp.int32),
        ],
    )(x, Wp, bp, Wg_pad)


LPW = CHUNK_B // 16


def _lane_elem(ids_v, lane, i):
    jv = lane * LPW + i
    return plsc.load_gather(
        ids_v, [jnp.right_shift(jv, 1), jnp.bitwise_and(jv, 1)])


def _hist(ids_v, lane, zero16):
    def hist_body(i, accs):
        v = _lane_elem(ids_v, lane, i)
        return tuple(accs[e] + jnp.where(v == e, 1, 0)
                     for e in range(N_EXPERTS))
    return lax.fori_loop(0, LPW, hist_body, (zero16,) * N_EXPERTS)


def _route_body(ei_hbm, cnths_hbm, h_hbm, pos_hbm, texp_hbm, hs_hbm,
                ids_v, pos_v, allcnt_v, texp_v, idx0_v, idx1_v, rows_v,
                sem):
    wid = _wid()
    base = wid * CHUNK_B
    lane = lax.iota(jnp.int32, 16)
    zero16 = jnp.zeros((16,), jnp.int32)

    pltpu.sync_copy(ei_hbm.at[pl.ds(wid * TOK_W, TOK_W)], ids_v)
    pltpu.sync_copy(cnths_hbm, allcnt_v)
    accs = _hist(ids_v, lane, zero16)

    widv = jnp.full((16,), wid, jnp.int32)
    tot = zero16
    mybase_cnt = zero16
    for w in range(NW):
        row = allcnt_v[w // 2, w % 2, pl.ds(0, 16)]
        tot = tot + row
        mybase_cnt = mybase_cnt + jnp.where(
            jnp.full((16,), w, jnp.int32) < widv, row, 0)

    end_scal = []
    lane_base = []
    gs_run = jnp.int32(0)
    for e in range(N_EXPERTS):
        tot_e = tot[e]
        pcnt_e = jnp.bitwise_and(tot_e + (GTILE - 1), ~(GTILE - 1))
        base_e = gs_run + mybase_cnt[e]
        gs_run = gs_run + pcnt_e
        end_scal.append(gs_run)
        vec = zero16
        run_s = base_e
        acc = accs[e]
        for l in range(16):
            vec = jnp.where(lane == l, jnp.full((16,), run_s, jnp.int32), vec)
            run_s = run_s + acc[l]
        lane_base.append(vec)

    def pos_body(i, rs):
        idxv = lane * LPW + i
        v = _lane_elem(ids_v, lane, i)
        posv = zero16
        new = []
        for e in range(N_EXPERTS):
            m = v == e
            posv = jnp.where(m, lane_base[e] + rs[e], posv)
            new.append(rs[e] + jnp.where(m, 1, 0))
        plsc.store_scatter(pos_v, [idxv], posv)
        return tuple(new)
    lax.fori_loop(0, LPW, pos_body, (zero16,) * N_EXPERTS)
    pltpu.sync_copy(pos_v, pos_hbm.at[pl.ds(base, CHUNK_B)])

    def dis_body(c, _):
        t0 = wid * TOK_W + c * TOKC
        a0 = c * TOKC * 2
        pltpu.sync_copy(h_hbm.at[pl.ds(t0, TOKC)], rows_v)
        for j in range(TOKC // 16):
            tl = jnp.full((16,), a0 + j * 32, jnp.int32) + lane * 2
            idx0_v[pl.ds(j * 16, 16)] = plsc.load_gather(pos_v, [tl])
            idx1_v[pl.ds(j * 16, 16)] = plsc.load_gather(pos_v, [tl + 1])
        cp0 = pltpu.async_copy(rows_v, hs_hbm.at[idx0_v], sem)
        cp1 = pltpu.async_copy(rows_v, hs_hbm.at[idx1_v], sem)
        cp0.wait()
        cp1.wait()
        return 0
    lax.fori_loop(0, TOK_W // TOKC, dis_body, 0)

    @pl.when(wid == 0)
    def _():
        for j in range(8):
            jv = (jnp.full((16,), j * 16, jnp.int32) + lane) * GTILE
            t = zero16
            for e in range(N_EXPERTS):
                t = t + jnp.where(
                    jv >= jnp.full((16,), end_scal[e], jnp.int32), 1, 0)
            texp_v[pl.ds(j * 16, 16)] = jnp.minimum(t, N_EXPERTS - 1)
        pltpu.sync_copy(texp_v, texp_hbm)


@jax.jit
def _stage_b2(ei, cnths, h_f):
    return pl.kernel(
        _route_body,
        mesh=_mesh,
        compiler_params=pltpu.CompilerParams(needs_layout_passes=False),
        out_type=[
            jax.ShapeDtypeStruct((A_TOT,), jnp.int32),
            jax.ShapeDtypeStruct((128,), jnp.int32),
            jax.ShapeDtypeStruct((PADTOT, ROWW), jnp.float32),
        ],
        scratch_types=[
            pltpu.VMEM((TOK_W, EPAD), jnp.int32),
            pltpu.VMEM((CHUNK_B,), jnp.int32),
            pltpu.VMEM((NW // 2, 2, EPAD), jnp.int32),
            pltpu.VMEM((128,), jnp.int32),
            pltpu.VMEM((TOKC,), jnp.int32),
            pltpu.VMEM((TOKC,), jnp.int32),
            pltpu.VMEM((TOKC, ROWW), jnp.float32),
            pltpu.SemaphoreType.DMA,
        ],
    )(ei, cnths, h_f)


def _ffn_body(texp_ref, hs_ref, w1_ref, b1_ref, w2_ref, b2_ref, ys_ref):
    hsb = hs_ref[...].astype(jnp.bfloat16)
    hid = lax.dot_general(hsb, w1_ref[0], _NT,
                          preferred_element_type=jnp.float32)
    hid = jnp.maximum(hid + b1_ref[0], 0.0)
    out = lax.dot_general(hid.astype(jnp.bfloat16), w2_ref[0], _NT,
                          preferred_element_type=jnp.float32)
    ys_ref[...] = out + b2_ref[0]


@jax.jit
def _stage_d(texp, hs_bf, W1b, b1, W2b, b2):
    grid_spec = pltpu.PrefetchScalarGridSpec(
        num_scalar_prefetch=1,
        grid=(NTILES,),
        in_specs=[
            pl.BlockSpec((GTILE, HIDDEN), lambda i, t: (i, 0)),
            pl.BlockSpec((1, EXPERT_DIM, HIDDEN), lambda i, t: (t[i], 0, 0)),
            pl.BlockSpec((1, 1, EXPERT_DIM), lambda i, t: (t[i], 0, 0)),
            pl.BlockSpec((1, HIDDEN, EXPERT_DIM), lambda i, t: (t[i], 0, 0)),
            pl.BlockSpec((1, 1, HIDDEN), lambda i, t: (t[i], 0, 0)),
        ],
        out_specs=pl.BlockSpec((GTILE, HIDDEN), lambda i, t: (i, 0)),
    )
    return pl.pallas_call(
        _ffn_body,
        grid_spec=grid_spec,
        out_shape=jax.ShapeDtypeStruct((PADTOT, HIDDEN), jnp.float32),
    )(texp, hs_bf, W1b, b1, W2b, b2)


def _combine_body(ys_hbm, pos_hbm, y0_hbm, y1_hbm, posc_v, idx0_v,
                  idx1_v, y0_v, y1_v, sem):
    wid = _wid()
    lane = lax.iota(jnp.int32, 16)

    def chunk_body(c, _):
        tok0 = wid * TOK_W + c * TCH
        pltpu.sync_copy(pos_hbm.at[pl.ds(tok0 * 2, TCH * 2)], posc_v)
        for j in range(TCH // 16):
            tl = jnp.full((16,), j * 16, jnp.int32) + lane
            idx0_v[pl.ds(j * 16, 16)] = plsc.load_gather(posc_v, [tl * 2])
            idx1_v[pl.ds(j * 16, 16)] = plsc.load_gather(posc_v, [tl * 2 + 1])
        cp0 = pltpu.async_copy(ys_hbm.at[idx0_v], y0_v, sem)
        cp1 = pltpu.async_copy(ys_hbm.at[idx1_v], y1_v, sem)
        cp0.wait()
        cp1.wait()
        cp2 = pltpu.async_copy(y0_v, y0_hbm.at[pl.ds(tok0, TCH)], sem)
        cp3 = pltpu.async_copy(y1_v, y1_hbm.at[pl.ds(tok0, TCH)], sem)
        cp2.wait()
        cp3.wait()
        return 0
    lax.fori_loop(0, TOK_W // TCH, chunk_body, 0)


@jax.jit
def _stage_e(ys, pos):
    return pl.kernel(
        _combine_body,
        mesh=_mesh,
        compiler_params=pltpu.CompilerParams(needs_layout_passes=False),
        out_type=[
            jax.ShapeDtypeStruct((TOKENS, ROWW), jnp.float32),
            jax.ShapeDtypeStruct((TOKENS, ROWW), jnp.float32),
        ],
        scratch_types=[
            pltpu.VMEM((TCH * 2,), jnp.int32),
            pltpu.VMEM((TCH,), jnp.int32),
            pltpu.VMEM((TCH,), jnp.int32),
            pltpu.VMEM((TCH, ROWW), jnp.float32),
            pltpu.VMEM((TCH, ROWW), jnp.float32),
            pltpu.SemaphoreType.DMA,
        ],
    )(ys, pos)


def _cls_body(y0_ref, y1_ref, wp_ref, wc_ref, bc_ref, cls_ref):
    w0 = wp_ref[:, 0:1]
    w1 = wp_ref[:, 1:2]
    h2 = jnp.maximum(y0_ref[...] * w0 + y1_ref[...] * w1, 0.0)
    cls = lax.dot_general(h2.astype(jnp.bfloat16), wc_ref[...], _NT,
                          preferred_element_type=jnp.float32)
    cls_ref[...] = cls + bc_ref[...]


@jax.jit
def _stage_f(y0, y1, wi_pad, Wcb, bc):
    full = lambda *shape: pl.BlockSpec(shape, lambda i: (0,) * len(shape))
    return pl.pallas_call(
        _cls_body,
        grid=(TOKENS // TM,),
        in_specs=[
            pl.BlockSpec((TM, HIDDEN), lambda i: (i, 0)),
            pl.BlockSpec((TM, HIDDEN), lambda i: (i, 0)),
            pl.BlockSpec((TM, EPAD), lambda i: (i, 0)),
            full(N_CLASSES, HIDDEN),
            full(1, N_CLASSES),
        ],
        out_specs=pl.BlockSpec((TM, N_CLASSES), lambda i: (i, 0)),
        out_shape=jax.ShapeDtypeStruct((TOKENS, N_CLASSES), jnp.float32),
    )(y0, y1, wi_pad, Wcb, bc)


def kernel(x, Wp, bp, Wg, W1, b1, W2, b2, Wc, bc):
    Wg_pad = jnp.zeros((EPAD, HIDDEN), jnp.float32).at[:N_EXPERTS].set(Wg)
    probs_pad, h_f, ei_pad, wi_pad, cnths = _stage_a(x, Wp, bp[None, :],
                                                     Wg_pad)
    pos, texp, hs = _stage_b2(ei_pad, cnths, h_f)
    ys = _stage_d(texp, hs, W1.astype(jnp.bfloat16),
                  b1.reshape(N_EXPERTS, 1, EXPERT_DIM),
                  W2.astype(jnp.bfloat16),
                  b2.reshape(N_EXPERTS, 1, HIDDEN))
    y0, y1 = _stage_e(ys, pos)
    cls = _stage_f(y0, y1, wi_pad, Wc.astype(jnp.bfloat16), bc[None, :])
    return cls, probs_pad[:, :N_EXPERTS]
